# Initial kernel scaffold; baseline (speedup 1.0000x reference)
import jax
import jax.numpy as jnp
from jax.experimental import pallas as pl


def kernel(feat0, feat1, feat2, nei0, nei1, W0, b0, W1, b1, W2, b2, att0, att1, fuse_W, fuse_b, fuse_att):
    raise NotImplementedError("write your pallas kernel here")


# speedup vs baseline: 7.4434x; 7.4434x over previous
import functools

import jax
import jax.numpy as jnp
from jax import lax
from jax.experimental import pallas as pl
from jax.experimental.pallas import tpu as pltpu
from jax.experimental.pallas import tpu_sc as plsc

N = 10000
D = 128
H = 64
S = 32

RB = 400
GRID = N // RB

C = 16
BROWS = C * S
NCHUNK = N // C
IDXW = 128
GPC = BROWS // IDXW
NEIROWS = N * S // IDXW


def _elu(x):
    return jnp.where(x > 0, x, jnp.exp(x) - 1.0)


def _proj_body(f0_r, f1_r, f2_r, w0_r, b0_r, w1_r, b1_r, w2_r, b2_r,
               a0_r, a1_r, h1_o, h2_o, t0_o, s1_o, t1_o, s2_o):
    def proj(f_r, w_r, b_r):
        h = lax.dot_general(f_r[...], w_r[...], (((1,), (1,)), ((), ())),
                            preferred_element_type=jnp.float32) + b_r[...]
        return _elu(h)

    h0 = proj(f0_r, w0_r, b0_r)
    h1 = proj(f1_r, w1_r, b1_r)
    h2 = proj(f2_r, w2_r, b2_r)
    h1_o[...] = h1
    h2_o[...] = h2
    att0 = a0_r[...]
    att1 = a1_r[...]

    def score(h, avec):
        return jnp.sum(h * avec[None, :], axis=1).reshape(1, 1, RB)

    t0_o[...] = score(h0, att0[0, :H])
    s1_o[...] = score(h1, att0[0, H:])
    t1_o[...] = score(h0, att1[0, :H])
    s2_o[...] = score(h2, att1[0, H:])


def _run_proj(feat0, feat1, feat2, W0, b0, W1, b1, W2, b2, att0, att1):
    row = pl.BlockSpec((RB, D), lambda i: (i, 0))
    full = lambda shape: pl.BlockSpec(shape, lambda i: tuple(0 for _ in shape))
    sc3 = pl.BlockSpec((1, 1, RB), lambda i: (i, 0, 0))
    outs = pl.pallas_call(
        _proj_body,
        grid=(GRID,),
        in_specs=[row, row, row,
                  full((H, D)), full((1, H)),
                  full((H, D)), full((1, H)),
                  full((H, D)), full((1, H)),
                  full((1, 2 * H)), full((1, 2 * H))],
        out_specs=[pl.BlockSpec((RB, H), lambda i: (i, 0)),
                   pl.BlockSpec((RB, H), lambda i: (i, 0)),
                   sc3, sc3, sc3, sc3],
        out_shape=[jax.ShapeDtypeStruct((N, H), jnp.float32),
                   jax.ShapeDtypeStruct((N, H), jnp.float32)] +
                  [jax.ShapeDtypeStruct((GRID, 1, RB), jnp.float32)] * 4,
    )(feat0, feat1, feat2, W0, b0.reshape(1, H), W1, b1.reshape(1, H),
      W2, b2.reshape(1, H), att0, att1)
    h1, h2, t0, s1, t1, s2 = outs
    flat = lambda x: x.reshape(N)
    return h1, h2, flat(t0), flat(s1), flat(t1), flat(s2)


def _sc_body(h1_hbm, h2_hbm, t0_hbm, s1_hbm, t1_hbm, s2_hbm,
             nei0_hbm, nei1_hbm, vs_hbm, vt_hbm,
             t0_v, s1_v, t1_v, s2_v, idx_v, rows_v, w_v, out_v, sem):
    info = plsc.get_sparse_core_info()
    nw = info.num_cores * info.num_subcores
    trips = (NCHUNK + nw - 1) // nw
    wid = lax.axis_index("s") * info.num_cores + lax.axis_index("c")

    pltpu.sync_copy(t0_hbm, t0_v)
    pltpu.sync_copy(s1_hbm, s1_v)
    pltpu.sync_copy(t1_hbm, t1_v)
    pltpu.sync_copy(s2_hbm, s2_v)

    for tbl_hbm, nei_hbm, tgt_v, src_v, out_hbm in (
            (h1_hbm, nei0_hbm, t0_v, s1_v, vs_hbm),
            (h2_hbm, nei1_hbm, t1_v, s2_v, vt_hbm)):

        def chunk_body(i, _, tbl_hbm=tbl_hbm, nei_hbm=nei_hbm, tgt_v=tgt_v,
                       src_v=src_v, out_hbm=out_hbm):
            cid = wid + i * nw

            @pl.when(cid < NCHUNK)
            def _():
                pltpu.sync_copy(nei_hbm.at[pl.ds(cid * GPC, GPC)], idx_v)
                handles = [
                    pltpu.async_copy(tbl_hbm.at[idx_v.at[g]],
                                     rows_v.at[pl.ds(g * IDXW, IDXW)], sem)
                    for g in range(GPC)
                ]
                node0 = cid * C
                tgt_chunk = tgt_v[pl.ds(node0, C)]

                for n in range(C):
                    r = n // (IDXW // S)
                    col = (n % (IDXW // S)) * S
                    iv0 = idx_v[r, pl.ds(col, 16)]
                    iv1 = idx_v[r, pl.ds(col + 16, 16)]
                    sv0 = plsc.load_gather(src_v, [iv0])
                    sv1 = plsc.load_gather(src_v, [iv1])
                    t = tgt_chunk[n]
                    a0 = t + sv0
                    a0 = jnp.where(a0 > 0, a0, a0 * 0.01)
                    a1 = t + sv1
                    a1 = jnp.where(a1 > 0, a1, a1 * 0.01)
                    m = jnp.maximum(jnp.max(a0), jnp.max(a1))
                    e0 = jnp.exp(a0 - m)
                    e1 = jnp.exp(a1 - m)
                    dv = jnp.full((16,), jnp.sum(e0) + jnp.sum(e1), jnp.float32)
                    rv = 1.0 / dv
                    w_v[pl.ds(n * S, 16)] = e0 * rv
                    w_v[pl.ds(n * S + 16, 16)] = e1 * rv

                for h in handles:
                    h.wait()

                def abody(n, _):
                    base = n * S
                    wv0 = w_v[pl.ds(base, 16)]
                    wv1 = w_v[pl.ds(base + 16, 16)]
                    accs = [jnp.zeros((16,), jnp.float32) for _ in range(H // 16)]
                    for s in range(S):
                        w = wv0[s] if s < 16 else wv1[s - 16]
                        for k in range(H // 16):
                            accs[k] = accs[k] + w * rows_v[base + s, pl.ds(k * 16, 16)]
                    for k in range(H // 16):
                        out_v[n, pl.ds(k * 16, 16)] = accs[k]
                    return 0

                lax.fori_loop(0, C, abody, 0)
                pltpu.sync_copy(out_v, out_hbm.at[pl.ds(node0, C)])

            return 0

        lax.fori_loop(0, trips, chunk_body, 0)


def _run_sc(h1, h2, t0, s1, t1, s2, nei0f, nei1f):
    mesh = plsc.VectorSubcoreMesh(core_axis_name="c", subcore_axis_name="s")
    fn = pl.kernel(
        _sc_body,
        mesh=mesh,
        compiler_params=pltpu.CompilerParams(needs_layout_passes=False,
                                             use_tc_tiling_on_sc=False),
        out_type=[jax.ShapeDtypeStruct((N, H), jnp.float32),
                  jax.ShapeDtypeStruct((N, H), jnp.float32)],
        scratch_types=[
            pltpu.VMEM((N,), jnp.float32),
            pltpu.VMEM((N,), jnp.float32),
            pltpu.VMEM((N,), jnp.float32),
            pltpu.VMEM((N,), jnp.float32),
            pltpu.VMEM((GPC, IDXW), jnp.int32),
            pltpu.VMEM((BROWS, H), jnp.float32),
            pltpu.VMEM((BROWS,), jnp.float32),
            pltpu.VMEM((C, H), jnp.float32),
            pltpu.SemaphoreType.DMA,
        ],
    )
    return fn(h1, h2, t0, s1, t1, s2, nei0f, nei1f)


def _fuse_body(vt_r, vs_r, fw_r, fb_r, et_o, es_o, sums_o):
    i = pl.program_id(0)
    et = _elu(vt_r[...])
    es = _elu(vs_r[...])
    et_o[...] = et
    es_o[...] = es

    def tproj(e):
        return jnp.tanh(lax.dot_general(e, fw_r[...], (((1,), (1,)), ((), ())),
                                        preferred_element_type=jnp.float32)
                        + fb_r[...])

    part = jnp.stack([jnp.sum(tproj(et), axis=0), jnp.sum(tproj(es), axis=0)])
    prev = jnp.where(i == 0, jnp.zeros_like(part), sums_o[...])
    sums_o[...] = prev + part


def _run_fuse(vt, vs, fuse_W, fuse_b):
    row = pl.BlockSpec((RB, H), lambda i: (i, 0))
    full = lambda shape: pl.BlockSpec(shape, lambda i: tuple(0 for _ in shape))
    return pl.pallas_call(
        _fuse_body,
        grid=(GRID,),
        in_specs=[row, row, full((H, H)), full((1, H))],
        out_specs=[row, row, pl.BlockSpec((2, H), lambda i: (0, 0))],
        out_shape=[jax.ShapeDtypeStruct((N, H), jnp.float32),
                   jax.ShapeDtypeStruct((N, H), jnp.float32),
                   jax.ShapeDtypeStruct((2, H), jnp.float32)],
    )(vt, vs, fuse_W, fuse_b.reshape(1, H))


def _comb_body(b_r, et_r, es_r, z_o):
    z_o[...] = b_r[0, 0] * et_r[...] + b_r[0, 1] * es_r[...]


def _run_combine(beta, et, es):
    row = pl.BlockSpec((RB, H), lambda i: (i, 0))
    return pl.pallas_call(
        _comb_body,
        grid=(GRID,),
        in_specs=[pl.BlockSpec(memory_space=pltpu.SMEM), row, row],
        out_specs=row,
        out_shape=jax.ShapeDtypeStruct((N, H), jnp.float32),
    )(beta.reshape(1, 2), et, es)


@jax.jit
def kernel(feat0, feat1, feat2, nei0, nei1, W0, b0, W1, b1, W2, b2,
           att0, att1, fuse_W, fuse_b, fuse_att):
    nei0f = jnp.reshape(nei0.astype(jnp.int32), (NEIROWS, IDXW))
    nei1f = jnp.reshape(nei1.astype(jnp.int32), (NEIROWS, IDXW))

    h1, h2, t0, s1, t1, s2 = _run_proj(
        feat0, feat1, feat2, W0, b0, W1, b1, W2, b2, att0, att1)

    vs, vt = _run_sc(h1, h2, t0, s1, t1, s2, nei0f, nei1f)

    et, es, sums = _run_fuse(vt, vs, fuse_W, fuse_b)

    sp = sums / N
    betas = sp @ fuse_att[0]
    beta = jax.nn.softmax(betas)

    return _run_combine(beta, et, es)


# speedup vs baseline: 8.8178x; 1.1846x over previous
import functools

import jax
import jax.numpy as jnp
from jax import lax
from jax.experimental import pallas as pl
from jax.experimental.pallas import tpu as pltpu
from jax.experimental.pallas import tpu_sc as plsc

N = 10000
D = 128
H = 64
S = 32

RB = 400
GRID = N // RB

C = 16
BROWS = C * S
NCHUNK = N // C
IDXW = 128
GPC = BROWS // IDXW
NEIROWS = N * S // IDXW


def _elu(x):
    return jnp.where(x > 0, x, jnp.exp(x) - 1.0)


def _proj_body(f0_r, f1_r, f2_r, w0_r, b0_r, w1_r, b1_r, w2_r, b2_r,
               a0_r, a1_r, h1_o, h2_o, t0_o, s1_o, t1_o, s2_o):
    def proj(f_r, w_r, b_r):
        h = lax.dot_general(f_r[...], w_r[...], (((1,), (1,)), ((), ())),
                            preferred_element_type=jnp.float32) + b_r[...]
        return _elu(h)

    h0 = proj(f0_r, w0_r, b0_r)
    h1 = proj(f1_r, w1_r, b1_r)
    h2 = proj(f2_r, w2_r, b2_r)
    h1_o[...] = h1
    h2_o[...] = h2
    att0 = a0_r[...]
    att1 = a1_r[...]

    def score(h, avec):
        return jnp.sum(h * avec[None, :], axis=1).reshape(1, 1, RB)

    t0_o[...] = score(h0, att0[0, :H])
    s1_o[...] = score(h1, att0[0, H:])
    t1_o[...] = score(h0, att1[0, :H])
    s2_o[...] = score(h2, att1[0, H:])


def _run_proj(feat0, feat1, feat2, W0, b0, W1, b1, W2, b2, att0, att1):
    row = pl.BlockSpec((RB, D), lambda i: (i, 0))
    full = lambda shape: pl.BlockSpec(shape, lambda i: tuple(0 for _ in shape))
    sc3 = pl.BlockSpec((1, 1, RB), lambda i: (i, 0, 0))
    outs = pl.pallas_call(
        _proj_body,
        grid=(GRID,),
        in_specs=[row, row, row,
                  full((H, D)), full((1, H)),
                  full((H, D)), full((1, H)),
                  full((H, D)), full((1, H)),
                  full((1, 2 * H)), full((1, 2 * H))],
        out_specs=[pl.BlockSpec((RB, H), lambda i: (i, 0)),
                   pl.BlockSpec((RB, H), lambda i: (i, 0)),
                   sc3, sc3, sc3, sc3],
        out_shape=[jax.ShapeDtypeStruct((N, H), jnp.float32),
                   jax.ShapeDtypeStruct((N, H), jnp.float32)] +
                  [jax.ShapeDtypeStruct((GRID, 1, RB), jnp.float32)] * 4,
    )(feat0, feat1, feat2, W0, b0.reshape(1, H), W1, b1.reshape(1, H),
      W2, b2.reshape(1, H), att0, att1)
    h1, h2, t0, s1, t1, s2 = outs
    flat = lambda x: x.reshape(N)
    return h1, h2, flat(t0), flat(s1), flat(t1), flat(s2)


NCMAX = 20


def _sc_body(h1_hbm, h2_hbm, t0_hbm, s1_hbm, t1_hbm, s2_hbm,
             nei0_hbm, nei1_hbm, vs_hbm, vt_hbm,
             tgt_v, src_v, idx_v, rows0_v, rows1_v, w_v, out0_v, out1_v,
             rsem0, rsem1, osem0, osem1):
    info = plsc.get_sparse_core_info()
    nw = info.num_cores * info.num_subcores
    wid = lax.axis_index("s") * info.num_cores + lax.axis_index("c")
    start = (wid * NCHUNK) // nw
    end = ((wid + 1) * NCHUNK) // nw
    nc = end - start

    for tbl_hbm, nei_hbm, tgt_hbm, src_hbm, out_hbm in (
            (h1_hbm, nei0_hbm, t0_hbm, s1_hbm, vs_hbm),
            (h2_hbm, nei1_hbm, t1_hbm, s2_hbm, vt_hbm)):

        pltpu.sync_copy(tgt_hbm, tgt_v)
        pltpu.sync_copy(src_hbm, src_v)
        pltpu.sync_copy(nei_hbm.at[pl.ds(start * GPC, NCMAX * GPC)], idx_v)

        def issue_gathers(j, rows_b, sem, tbl_hbm=tbl_hbm):
            for g in range(GPC):
                pltpu.async_copy(tbl_hbm.at[idx_v.at[j * GPC + g]],
                                 rows_b.at[pl.ds(g * IDXW, IDXW)], sem)

        def wait_gathers(j, rows_b, sem, tbl_hbm=tbl_hbm):
            for g in range(GPC):
                pltpu.make_async_copy(tbl_hbm.at[idx_v.at[j * GPC + g]],
                                      rows_b.at[pl.ds(g * IDXW, IDXW)],
                                      sem).wait()

        def process(j, j2, rows_b, sem, out_b, osem,
                    out_hbm=out_hbm, issue_gathers=issue_gathers,
                    wait_gathers=wait_gathers):
            cid = start + j
            node0 = cid * C
            tgt_chunk = tgt_v[pl.ds(node0, C)]

            for n in range(C):
                r = j * GPC + n // (IDXW // S)
                col = (n % (IDXW // S)) * S
                iv0 = idx_v[r, pl.ds(col, 16)]
                iv1 = idx_v[r, pl.ds(col + 16, 16)]
                sv0 = plsc.load_gather(src_v, [iv0])
                sv1 = plsc.load_gather(src_v, [iv1])
                t = tgt_chunk[n]
                a0 = t + sv0
                a0 = jnp.where(a0 > 0, a0, a0 * 0.01)
                a1 = t + sv1
                a1 = jnp.where(a1 > 0, a1, a1 * 0.01)
                m = jnp.maximum(jnp.max(a0), jnp.max(a1))
                e0 = jnp.exp(a0 - m)
                e1 = jnp.exp(a1 - m)
                dv = jnp.full((16,), jnp.sum(e0) + jnp.sum(e1), jnp.float32)
                rv = 1.0 / dv
                w_v[pl.ds(n * S, 16)] = e0 * rv
                w_v[pl.ds(n * S + 16, 16)] = e1 * rv

            @pl.when(j2 > 0)
            def _():
                pltpu.make_async_copy(
                    out_b, out_hbm.at[pl.ds((cid - 2) * C, C)], osem).wait()

            wait_gathers(j, rows_b, sem)

            def abody(n, _):
                base = n * S
                wv0 = w_v[pl.ds(base, 16)]
                wv1 = w_v[pl.ds(base + 16, 16)]
                accs = [jnp.zeros((16,), jnp.float32) for _ in range(H // 16)]
                for s in range(S):
                    w = wv0[s] if s < 16 else wv1[s - 16]
                    for k in range(H // 16):
                        accs[k] = accs[k] + w * rows_b[base + s, pl.ds(k * 16, 16)]
                for k in range(H // 16):
                    out_b[n, pl.ds(k * 16, 16)] = accs[k]
                return 0

            lax.fori_loop(0, C, abody, 0)
            pltpu.async_copy(out_b, out_hbm.at[pl.ds(node0, C)], osem)

            @pl.when(j + 2 < nc)
            def _():
                issue_gathers(j + 2, rows_b, sem)

        issue_gathers(0, rows0_v, rsem0)
        issue_gathers(1, rows1_v, rsem1)

        def pair_body(j2, _):
            process(2 * j2, j2, rows0_v, rsem0, out0_v, osem0)

            @pl.when(2 * j2 + 1 < nc)
            def _():
                process(2 * j2 + 1, j2, rows1_v, rsem1, out1_v, osem1)

            return 0

        lax.fori_loop(0, (nc + 1) // 2, pair_body, 0)

        je = ((nc - 1) // 2) * 2
        jo = ((nc - 2) // 2) * 2 + 1
        pltpu.make_async_copy(
            out0_v, out_hbm.at[pl.ds((start + je) * C, C)], osem0).wait()
        pltpu.make_async_copy(
            out1_v, out_hbm.at[pl.ds((start + jo) * C, C)], osem1).wait()


def _run_sc(h1, h2, t0, s1, t1, s2, nei0f, nei1f):
    mesh = plsc.VectorSubcoreMesh(core_axis_name="c", subcore_axis_name="s")
    fn = pl.kernel(
        _sc_body,
        mesh=mesh,
        compiler_params=pltpu.CompilerParams(needs_layout_passes=False,
                                             use_tc_tiling_on_sc=False),
        out_type=[jax.ShapeDtypeStruct((N, H), jnp.float32),
                  jax.ShapeDtypeStruct((N, H), jnp.float32)],
        scratch_types=[
            pltpu.VMEM((N,), jnp.float32),
            pltpu.VMEM((N,), jnp.float32),
            pltpu.VMEM((NCMAX * GPC, IDXW), jnp.int32),
            pltpu.VMEM((BROWS, H), jnp.float32),
            pltpu.VMEM((BROWS, H), jnp.float32),
            pltpu.VMEM((BROWS,), jnp.float32),
            pltpu.VMEM((C, H), jnp.float32),
            pltpu.VMEM((C, H), jnp.float32),
            pltpu.SemaphoreType.DMA,
            pltpu.SemaphoreType.DMA,
            pltpu.SemaphoreType.DMA,
            pltpu.SemaphoreType.DMA,
        ],
    )
    return fn(h1, h2, t0, s1, t1, s2, nei0f, nei1f)


def _fuse_body(vt_r, vs_r, fw_r, fb_r, et_o, es_o, sums_o):
    i = pl.program_id(0)
    et = _elu(vt_r[...])
    es = _elu(vs_r[...])
    et_o[...] = et
    es_o[...] = es

    def tproj(e):
        return jnp.tanh(lax.dot_general(e, fw_r[...], (((1,), (1,)), ((), ())),
                                        preferred_element_type=jnp.float32)
                        + fb_r[...])

    part = jnp.stack([jnp.sum(tproj(et), axis=0), jnp.sum(tproj(es), axis=0)])
    prev = jnp.where(i == 0, jnp.zeros_like(part), sums_o[...])
    sums_o[...] = prev + part


def _run_fuse(vt, vs, fuse_W, fuse_b):
    row = pl.BlockSpec((RB, H), lambda i: (i, 0))
    full = lambda shape: pl.BlockSpec(shape, lambda i: tuple(0 for _ in shape))
    return pl.pallas_call(
        _fuse_body,
        grid=(GRID,),
        in_specs=[row, row, full((H, H)), full((1, H))],
        out_specs=[row, row, pl.BlockSpec((2, H), lambda i: (0, 0))],
        out_shape=[jax.ShapeDtypeStruct((N, H), jnp.float32),
                   jax.ShapeDtypeStruct((N, H), jnp.float32),
                   jax.ShapeDtypeStruct((2, H), jnp.float32)],
    )(vt, vs, fuse_W, fuse_b.reshape(1, H))


def _comb_body(b_r, et_r, es_r, z_o):
    z_o[...] = b_r[0, 0] * et_r[...] + b_r[0, 1] * es_r[...]


def _run_combine(beta, et, es):
    row = pl.BlockSpec((RB, H), lambda i: (i, 0))
    return pl.pallas_call(
        _comb_body,
        grid=(GRID,),
        in_specs=[pl.BlockSpec(memory_space=pltpu.SMEM), row, row],
        out_specs=row,
        out_shape=jax.ShapeDtypeStruct((N, H), jnp.float32),
    )(beta.reshape(1, 2), et, es)


@jax.jit
def kernel(feat0, feat1, feat2, nei0, nei1, W0, b0, W1, b1, W2, b2,
           att0, att1, fuse_W, fuse_b, fuse_att):
    nei0f = jnp.reshape(nei0.astype(jnp.int32), (NEIROWS, IDXW))
    nei1f = jnp.reshape(nei1.astype(jnp.int32), (NEIROWS, IDXW))

    h1, h2, t0, s1, t1, s2 = _run_proj(
        feat0, feat1, feat2, W0, b0, W1, b1, W2, b2, att0, att1)

    vs, vt = _run_sc(h1, h2, t0, s1, t1, s2, nei0f, nei1f)

    et, es, sums = _run_fuse(vt, vs, fuse_W, fuse_b)

    sp = sums / N
    betas = sp @ fuse_att[0]
    beta = jax.nn.softmax(betas)

    return _run_combine(beta, et, es)


# speedup vs baseline: 8.9141x; 1.0109x over previous
import functools

import jax
import jax.numpy as jnp
from jax import lax
from jax.experimental import pallas as pl
from jax.experimental.pallas import tpu as pltpu
from jax.experimental.pallas import tpu_sc as plsc

N = 10000
D = 128
H = 64
S = 32

RB = 400
GRID = N // RB

C = 16
BROWS = C * S
NCHUNK = N // C
IDXW = 128
GPC = BROWS // IDXW
NEIROWS = N * S // IDXW


def _elu(x):
    return jnp.where(x > 0, x, jnp.exp(x) - 1.0)


def _proj_body(f0_r, f1_r, f2_r, w0_r, b0_r, w1_r, b1_r, w2_r, b2_r,
               a0_r, a1_r, h1_o, h2_o, t0_o, s1_o, t1_o, s2_o):
    def proj(f_r, w_r, b_r):
        h = lax.dot_general(f_r[...], w_r[...], (((1,), (1,)), ((), ())),
                            preferred_element_type=jnp.float32) + b_r[...]
        return _elu(h)

    h0 = proj(f0_r, w0_r, b0_r)
    h1 = proj(f1_r, w1_r, b1_r)
    h2 = proj(f2_r, w2_r, b2_r)
    h1_o[...] = h1
    h2_o[...] = h2
    att0 = a0_r[...]
    att1 = a1_r[...]

    def score(h, avec):
        return jnp.sum(h * avec[None, :], axis=1).reshape(1, 1, RB)

    t0_o[...] = score(h0, att0[0, :H])
    s1_o[...] = score(h1, att0[0, H:])
    t1_o[...] = score(h0, att1[0, :H])
    s2_o[...] = score(h2, att1[0, H:])


def _run_proj(feat0, feat1, feat2, W0, b0, W1, b1, W2, b2, att0, att1):
    row = pl.BlockSpec((RB, D), lambda i: (i, 0))
    full = lambda shape: pl.BlockSpec(shape, lambda i: tuple(0 for _ in shape))
    sc3 = pl.BlockSpec((1, 1, RB), lambda i: (i, 0, 0))
    outs = pl.pallas_call(
        _proj_body,
        grid=(GRID,),
        in_specs=[row, row, row,
                  full((H, D)), full((1, H)),
                  full((H, D)), full((1, H)),
                  full((H, D)), full((1, H)),
                  full((1, 2 * H)), full((1, 2 * H))],
        out_specs=[pl.BlockSpec((RB, H), lambda i: (i, 0)),
                   pl.BlockSpec((RB, H), lambda i: (i, 0)),
                   sc3, sc3, sc3, sc3],
        out_shape=[jax.ShapeDtypeStruct((N, H), jnp.float32),
                   jax.ShapeDtypeStruct((N, H), jnp.float32)] +
                  [jax.ShapeDtypeStruct((GRID, 1, RB), jnp.float32)] * 4,
    )(feat0, feat1, feat2, W0, b0.reshape(1, H), W1, b1.reshape(1, H),
      W2, b2.reshape(1, H), att0, att1)
    h1, h2, t0, s1, t1, s2 = outs
    flat = lambda x: x.reshape(N)
    return h1, h2, flat(t0), flat(s1), flat(t1), flat(s2)


NCMAX = 20


def _sc_body(h1_hbm, h2_hbm, t0_hbm, s1_hbm, t1_hbm, s2_hbm,
             nei0_hbm, nei1_hbm, vs_hbm, vt_hbm,
             tgt_v, src_v, idx_v, rows0_v, rows1_v, w_v, out0_v, out1_v,
             rsem0, rsem1, osem0, osem1):
    info = plsc.get_sparse_core_info()
    nw = info.num_cores * info.num_subcores
    wid = lax.axis_index("s") * info.num_cores + lax.axis_index("c")
    start = (wid * NCHUNK) // nw
    end = ((wid + 1) * NCHUNK) // nw
    nc = end - start

    for tbl_hbm, nei_hbm, tgt_hbm, src_hbm, out_hbm in (
            (h1_hbm, nei0_hbm, t0_hbm, s1_hbm, vs_hbm),
            (h2_hbm, nei1_hbm, t1_hbm, s2_hbm, vt_hbm)):

        pltpu.sync_copy(tgt_hbm, tgt_v)
        pltpu.sync_copy(src_hbm, src_v)
        pltpu.sync_copy(nei_hbm.at[pl.ds(start * GPC, NCMAX * GPC)], idx_v)

        def issue_gathers(j, rows_b, sem, tbl_hbm=tbl_hbm):
            for g in range(GPC):
                pltpu.async_copy(tbl_hbm.at[idx_v.at[j * GPC + g]],
                                 rows_b.at[pl.ds(g * IDXW, IDXW)], sem)

        def wait_gathers(j, rows_b, sem, tbl_hbm=tbl_hbm):
            for g in range(GPC):
                pltpu.make_async_copy(tbl_hbm.at[idx_v.at[j * GPC + g]],
                                      rows_b.at[pl.ds(g * IDXW, IDXW)],
                                      sem).wait()

        def process(j, j2, rows_b, sem, out_b, osem,
                    out_hbm=out_hbm, issue_gathers=issue_gathers,
                    wait_gathers=wait_gathers):
            cid = start + j
            node0 = cid * C
            tgt_chunk = tgt_v[pl.ds(node0, C)]

            for n in range(C):
                r = j * GPC + n // (IDXW // S)
                col = (n % (IDXW // S)) * S
                iv0 = idx_v[r, pl.ds(col, 16)]
                iv1 = idx_v[r, pl.ds(col + 16, 16)]
                sv0 = plsc.load_gather(src_v, [iv0])
                sv1 = plsc.load_gather(src_v, [iv1])
                t = tgt_chunk[n]
                a0 = t + sv0
                a0 = jnp.where(a0 > 0, a0, a0 * 0.01)
                a1 = t + sv1
                a1 = jnp.where(a1 > 0, a1, a1 * 0.01)
                m = jnp.maximum(jnp.max(a0), jnp.max(a1))
                e0 = jnp.exp(a0 - m)
                e1 = jnp.exp(a1 - m)
                dv = jnp.full((16,), jnp.sum(e0) + jnp.sum(e1), jnp.float32)
                rv = 1.0 / dv
                w_v[pl.ds(n * S, 16)] = e0 * rv
                w_v[pl.ds(n * S + 16, 16)] = e1 * rv

            @pl.when(j2 > 0)
            def _():
                pltpu.make_async_copy(
                    out_b, out_hbm.at[pl.ds((cid - 2) * C, C)], osem).wait()

            wait_gathers(j, rows_b, sem)

            def abody(n, _):
                base = n * S
                wv0 = w_v[pl.ds(base, 16)]
                wv1 = w_v[pl.ds(base + 16, 16)]
                accs = [jnp.zeros((16,), jnp.float32) for _ in range(H // 16)]
                for s in range(S):
                    w = wv0[s] if s < 16 else wv1[s - 16]
                    for k in range(H // 16):
                        accs[k] = accs[k] + w * rows_b[base + s, pl.ds(k * 16, 16)]
                for k in range(H // 16):
                    out_b[n, pl.ds(k * 16, 16)] = accs[k]
                return 0

            lax.fori_loop(0, C, abody, 0)
            pltpu.async_copy(out_b, out_hbm.at[pl.ds(node0, C)], osem)

            @pl.when(j + 2 < nc)
            def _():
                issue_gathers(j + 2, rows_b, sem)

        issue_gathers(0, rows0_v, rsem0)
        issue_gathers(1, rows1_v, rsem1)

        def pair_body(j2, _):
            process(2 * j2, j2, rows0_v, rsem0, out0_v, osem0)

            @pl.when(2 * j2 + 1 < nc)
            def _():
                process(2 * j2 + 1, j2, rows1_v, rsem1, out1_v, osem1)

            return 0

        lax.fori_loop(0, (nc + 1) // 2, pair_body, 0)

        je = ((nc - 1) // 2) * 2
        jo = ((nc - 2) // 2) * 2 + 1
        pltpu.make_async_copy(
            out0_v, out_hbm.at[pl.ds((start + je) * C, C)], osem0).wait()
        pltpu.make_async_copy(
            out1_v, out_hbm.at[pl.ds((start + jo) * C, C)], osem1).wait()


def _run_sc(h1, h2, t0, s1, t1, s2, nei0f, nei1f):
    mesh = plsc.VectorSubcoreMesh(core_axis_name="c", subcore_axis_name="s")
    fn = pl.kernel(
        _sc_body,
        mesh=mesh,
        compiler_params=pltpu.CompilerParams(needs_layout_passes=False,
                                             use_tc_tiling_on_sc=False),
        out_type=[jax.ShapeDtypeStruct((N, H), jnp.float32),
                  jax.ShapeDtypeStruct((N, H), jnp.float32)],
        scratch_types=[
            pltpu.VMEM((N,), jnp.float32),
            pltpu.VMEM((N,), jnp.float32),
            pltpu.VMEM((NCMAX * GPC, IDXW), jnp.int32),
            pltpu.VMEM((BROWS, H), jnp.float32),
            pltpu.VMEM((BROWS, H), jnp.float32),
            pltpu.VMEM((BROWS,), jnp.float32),
            pltpu.VMEM((C, H), jnp.float32),
            pltpu.VMEM((C, H), jnp.float32),
            pltpu.SemaphoreType.DMA,
            pltpu.SemaphoreType.DMA,
            pltpu.SemaphoreType.DMA,
            pltpu.SemaphoreType.DMA,
        ],
    )
    return fn(h1, h2, t0, s1, t1, s2, nei0f, nei1f)


def _fuse_body(vt_r, vs_r, fw_r, fb_r, fa_r, et_o, es_o, beta_o, sums):
    i = pl.program_id(0)
    et = _elu(vt_r[...])
    es = _elu(vs_r[...])
    et_o[...] = et
    es_o[...] = es

    def tproj(e):
        return jnp.tanh(lax.dot_general(e, fw_r[...], (((1,), (1,)), ((), ())),
                                        preferred_element_type=jnp.float32)
                        + fb_r[...])

    part = jnp.stack([jnp.sum(tproj(et), axis=0), jnp.sum(tproj(es), axis=0)])
    prev = jnp.where(i == 0, jnp.zeros_like(part), sums[...])
    sums[...] = prev + part

    @pl.when(i == GRID - 1)
    def _():
        sp = sums[...] * (1.0 / N)
        b2 = jnp.sum(sp * fa_r[...], axis=1, keepdims=True)
        e = jnp.exp(b2 - jnp.max(b2))
        beta_o[...] = e / jnp.sum(e)


def _run_fuse(vt, vs, fuse_W, fuse_b, fuse_att):
    row = pl.BlockSpec((RB, H), lambda i: (i, 0))
    full = lambda shape: pl.BlockSpec(shape, lambda i: tuple(0 for _ in shape))
    return pl.pallas_call(
        _fuse_body,
        grid=(GRID,),
        in_specs=[row, row, full((H, H)), full((1, H)), full((1, H))],
        out_specs=[row, row, pl.BlockSpec((2, 1), lambda i: (0, 0))],
        out_shape=[jax.ShapeDtypeStruct((N, H), jnp.float32),
                   jax.ShapeDtypeStruct((N, H), jnp.float32),
                   jax.ShapeDtypeStruct((2, 1), jnp.float32)],
        scratch_shapes=[pltpu.VMEM((2, H), jnp.float32)],
    )(vt, vs, fuse_W, fuse_b.reshape(1, H), fuse_att)


def _comb_body(b_r, et_r, es_r, z_o):
    z_o[...] = b_r[0] * et_r[...] + b_r[1] * es_r[...]


def _run_combine(beta, et, es):
    row = pl.BlockSpec((RB, H), lambda i: (i, 0))
    return pl.pallas_call(
        _comb_body,
        grid=(GRID,),
        in_specs=[pl.BlockSpec(memory_space=pltpu.SMEM), row, row],
        out_specs=row,
        out_shape=jax.ShapeDtypeStruct((N, H), jnp.float32),
    )(beta.reshape(2), et, es)


@jax.jit
def kernel(feat0, feat1, feat2, nei0, nei1, W0, b0, W1, b1, W2, b2,
           att0, att1, fuse_W, fuse_b, fuse_att):
    nei0f = jnp.reshape(nei0.astype(jnp.int32), (NEIROWS, IDXW))
    nei1f = jnp.reshape(nei1.astype(jnp.int32), (NEIROWS, IDXW))

    h1, h2, t0, s1, t1, s2 = _run_proj(
        feat0, feat1, feat2, W0, b0, W1, b1, W2, b2, att0, att1)

    vs, vt = _run_sc(h1, h2, t0, s1, t1, s2, nei0f, nei1f)

    et, es, beta = _run_fuse(vt, vs, fuse_W, fuse_b, fuse_att)

    return _run_combine(beta, et, es)


# speedup vs baseline: 10.2694x; 1.1520x over previous
import functools

import jax
import jax.numpy as jnp
from jax import lax
from jax.experimental import pallas as pl
from jax.experimental.pallas import tpu as pltpu
from jax.experimental.pallas import tpu_sc as plsc

N = 10000
D = 128
H = 64
S = 32

RB = 2000
GRID = N // RB
WOUT = 128

C = 16
BROWS = C * S
NCHUNK = N // C
IDXW = 128
GPC = BROWS // IDXW
NEIROWS = N * S // IDXW


def _elu(x):
    return jnp.where(x > 0, x, jnp.exp(x) - 1.0)


def _proj_body(f0_r, f1_r, f2_r, w0_r, b0_r, w1_r, b1_r, w2_r, b2_r,
               a0_r, a1_r, h1_o, h2_o, t0_o, s1_o, t1_o, s2_o):
    def proj(f_r, w_r, b_r):
        h = lax.dot_general(f_r[...], w_r[...], (((1,), (1,)), ((), ())),
                            preferred_element_type=jnp.float32) + b_r[...]
        return _elu(h)

    h0 = proj(f0_r, w0_r, b0_r)
    h1 = proj(f1_r, w1_r, b1_r)
    h2 = proj(f2_r, w2_r, b2_r)
    h1_o[...] = h1
    h2_o[...] = h2
    att0 = a0_r[...]
    att1 = a1_r[...]

    def score(h, avec):
        return jnp.sum(h * avec[None, :], axis=1).reshape(1, 1, RB)

    t0_o[...] = score(h0, att0[0, :H])
    s1_o[...] = score(h1, att0[0, H:])
    t1_o[...] = score(h0, att1[0, :H])
    s2_o[...] = score(h2, att1[0, H:])


def _run_proj(feat0, feat1, feat2, W0, b0, W1, b1, W2, b2, att0, att1):
    row = pl.BlockSpec((RB, D), lambda i: (i, 0))
    full = lambda shape: pl.BlockSpec(shape, lambda i: tuple(0 for _ in shape))
    sc3 = pl.BlockSpec((1, 1, RB), lambda i: (i, 0, 0))
    outs = pl.pallas_call(
        _proj_body,
        grid=(GRID,),
        in_specs=[row, row, row,
                  full((H, D)), full((1, H)),
                  full((H, D)), full((1, H)),
                  full((H, D)), full((1, H)),
                  full((1, 2 * H)), full((1, 2 * H))],
        out_specs=[pl.BlockSpec((RB, H), lambda i: (i, 0)),
                   pl.BlockSpec((RB, H), lambda i: (i, 0)),
                   sc3, sc3, sc3, sc3],
        out_shape=[jax.ShapeDtypeStruct((N, H), jnp.float32),
                   jax.ShapeDtypeStruct((N, H), jnp.float32)] +
                  [jax.ShapeDtypeStruct((GRID, 1, RB), jnp.float32)] * 4,
    )(feat0, feat1, feat2, W0, b0.reshape(1, H), W1, b1.reshape(1, H),
      W2, b2.reshape(1, H), att0, att1)
    h1, h2, t0, s1, t1, s2 = outs
    flat = lambda x: x.reshape(N)
    return h1, h2, flat(t0), flat(s1), flat(t1), flat(s2)


NCMAX = 20


def _sc_body(h1_hbm, h2_hbm, t0_hbm, s1_hbm, t1_hbm, s2_hbm,
             nei0_hbm, nei1_hbm, vs_hbm, vt_hbm,
             tgt_v, src_v, idx_v, rows0_v, rows1_v, w_v, out0_v, out1_v,
             rsem0, rsem1, osem0, osem1):
    info = plsc.get_sparse_core_info()
    nw = info.num_cores * info.num_subcores
    wid = lax.axis_index("s") * info.num_cores + lax.axis_index("c")
    start = (wid * NCHUNK) // nw
    end = ((wid + 1) * NCHUNK) // nw
    nc = end - start

    for tbl_hbm, nei_hbm, tgt_hbm, src_hbm, out_hbm in (
            (h1_hbm, nei0_hbm, t0_hbm, s1_hbm, vs_hbm),
            (h2_hbm, nei1_hbm, t1_hbm, s2_hbm, vt_hbm)):

        pltpu.sync_copy(tgt_hbm, tgt_v)
        pltpu.sync_copy(src_hbm, src_v)
        pltpu.sync_copy(nei_hbm.at[pl.ds(start * GPC, NCMAX * GPC)], idx_v)

        def issue_gathers(j, rows_b, sem, tbl_hbm=tbl_hbm):
            for g in range(GPC):
                pltpu.async_copy(tbl_hbm.at[idx_v.at[j * GPC + g]],
                                 rows_b.at[pl.ds(g * IDXW, IDXW)], sem)

        def wait_gathers(j, rows_b, sem, tbl_hbm=tbl_hbm):
            for g in range(GPC):
                pltpu.make_async_copy(tbl_hbm.at[idx_v.at[j * GPC + g]],
                                      rows_b.at[pl.ds(g * IDXW, IDXW)],
                                      sem).wait()

        def process(j, j2, rows_b, sem, out_b, osem,
                    out_hbm=out_hbm, issue_gathers=issue_gathers,
                    wait_gathers=wait_gathers):
            cid = start + j
            node0 = cid * C
            tgt_chunk = tgt_v[pl.ds(node0, C)]

            for n in range(C):
                r = j * GPC + n // (IDXW // S)
                col = (n % (IDXW // S)) * S
                iv0 = idx_v[r, pl.ds(col, 16)]
                iv1 = idx_v[r, pl.ds(col + 16, 16)]
                sv0 = plsc.load_gather(src_v, [iv0])
                sv1 = plsc.load_gather(src_v, [iv1])
                t = tgt_chunk[n]
                a0 = t + sv0
                a0 = jnp.where(a0 > 0, a0, a0 * 0.01)
                a1 = t + sv1
                a1 = jnp.where(a1 > 0, a1, a1 * 0.01)
                m = jnp.maximum(jnp.max(a0), jnp.max(a1))
                e0 = jnp.exp(a0 - m)
                e1 = jnp.exp(a1 - m)
                dv = jnp.full((16,), jnp.sum(e0) + jnp.sum(e1), jnp.float32)
                rv = 1.0 / dv
                w_v[pl.ds(n * S, 16)] = e0 * rv
                w_v[pl.ds(n * S + 16, 16)] = e1 * rv

            @pl.when(j2 > 0)
            def _():
                pltpu.make_async_copy(
                    out_b, out_hbm.at[pl.ds((cid - 2) * C, C)], osem).wait()

            wait_gathers(j, rows_b, sem)

            def abody(n, _):
                base = n * S
                wv0 = w_v[pl.ds(base, 16)]
                wv1 = w_v[pl.ds(base + 16, 16)]
                accs = [jnp.zeros((16,), jnp.float32) for _ in range(H // 16)]
                for s in range(S):
                    w = wv0[s] if s < 16 else wv1[s - 16]
                    for k in range(H // 16):
                        accs[k] = accs[k] + w * rows_b[base + s, pl.ds(k * 16, 16)]
                for k in range(H // 16):
                    out_b[n, pl.ds(k * 16, 16)] = accs[k]
                return 0

            lax.fori_loop(0, C, abody, 0)
            pltpu.async_copy(out_b, out_hbm.at[pl.ds(node0, C)], osem)

            @pl.when(j + 2 < nc)
            def _():
                issue_gathers(j + 2, rows_b, sem)

        issue_gathers(0, rows0_v, rsem0)
        issue_gathers(1, rows1_v, rsem1)

        def pair_body(j2, _):
            process(2 * j2, j2, rows0_v, rsem0, out0_v, osem0)

            @pl.when(2 * j2 + 1 < nc)
            def _():
                process(2 * j2 + 1, j2, rows1_v, rsem1, out1_v, osem1)

            return 0

        lax.fori_loop(0, (nc + 1) // 2, pair_body, 0)

        je = ((nc - 1) // 2) * 2
        jo = ((nc - 2) // 2) * 2 + 1
        pltpu.make_async_copy(
            out0_v, out_hbm.at[pl.ds((start + je) * C, C)], osem0).wait()
        pltpu.make_async_copy(
            out1_v, out_hbm.at[pl.ds((start + jo) * C, C)], osem1).wait()


def _run_sc(h1, h2, t0, s1, t1, s2, nei0f, nei1f):
    mesh = plsc.VectorSubcoreMesh(core_axis_name="c", subcore_axis_name="s")
    fn = pl.kernel(
        _sc_body,
        mesh=mesh,
        compiler_params=pltpu.CompilerParams(needs_layout_passes=False,
                                             use_tc_tiling_on_sc=False),
        out_type=[jax.ShapeDtypeStruct((N, WOUT), jnp.float32),
                  jax.ShapeDtypeStruct((N, WOUT), jnp.float32)],
        scratch_types=[
            pltpu.VMEM((N,), jnp.float32),
            pltpu.VMEM((N,), jnp.float32),
            pltpu.VMEM((NCMAX * GPC, IDXW), jnp.int32),
            pltpu.VMEM((BROWS, H), jnp.float32),
            pltpu.VMEM((BROWS, H), jnp.float32),
            pltpu.VMEM((BROWS,), jnp.float32),
            pltpu.VMEM((C, WOUT), jnp.float32),
            pltpu.VMEM((C, WOUT), jnp.float32),
            pltpu.SemaphoreType.DMA,
            pltpu.SemaphoreType.DMA,
            pltpu.SemaphoreType.DMA,
            pltpu.SemaphoreType.DMA,
        ],
    )
    return fn(h1, h2, t0, s1, t1, s2, nei0f, nei1f)


def _fuse_body(vt_r, vs_r, fw_r, fb_r, fa_r, et_o, es_o, beta_o, sums):
    i = pl.program_id(0)
    et = _elu(vt_r[:, :H])
    es = _elu(vs_r[:, :H])
    et_o[...] = et
    es_o[...] = es

    def tproj(e):
        return jnp.tanh(lax.dot_general(e, fw_r[...], (((1,), (1,)), ((), ())),
                                        preferred_element_type=jnp.float32)
                        + fb_r[...])

    part = jnp.stack([jnp.sum(tproj(et), axis=0), jnp.sum(tproj(es), axis=0)])
    prev = jnp.where(i == 0, jnp.zeros_like(part), sums[...])
    sums[...] = prev + part

    @pl.when(i == GRID - 1)
    def _():
        sp = sums[...] * (1.0 / N)
        b2 = jnp.sum(sp * fa_r[...], axis=1, keepdims=True)
        e = jnp.exp(b2 - jnp.max(b2))
        beta_o[...] = e / jnp.sum(e)


def _run_fuse(vt, vs, fuse_W, fuse_b, fuse_att):
    row = pl.BlockSpec((RB, H), lambda i: (i, 0))
    roww = pl.BlockSpec((RB, WOUT), lambda i: (i, 0))
    full = lambda shape: pl.BlockSpec(shape, lambda i: tuple(0 for _ in shape))
    return pl.pallas_call(
        _fuse_body,
        grid=(GRID,),
        in_specs=[roww, roww, full((H, H)), full((1, H)), full((1, H))],
        out_specs=[row, row, pl.BlockSpec((2, 1), lambda i: (0, 0))],
        out_shape=[jax.ShapeDtypeStruct((N, H), jnp.float32),
                   jax.ShapeDtypeStruct((N, H), jnp.float32),
                   jax.ShapeDtypeStruct((2, 1), jnp.float32)],
        scratch_shapes=[pltpu.VMEM((2, H), jnp.float32)],
    )(vt, vs, fuse_W, fuse_b.reshape(1, H), fuse_att)


def _comb_body(b_r, et_r, es_r, z_o):
    z_o[...] = b_r[0] * et_r[...] + b_r[1] * es_r[...]


def _run_combine(beta, et, es):
    row = pl.BlockSpec((RB, H), lambda i: (i, 0))
    return pl.pallas_call(
        _comb_body,
        grid=(GRID,),
        in_specs=[pl.BlockSpec(memory_space=pltpu.SMEM), row, row],
        out_specs=row,
        out_shape=jax.ShapeDtypeStruct((N, H), jnp.float32),
    )(beta.reshape(2), et, es)


@jax.jit
def kernel(feat0, feat1, feat2, nei0, nei1, W0, b0, W1, b1, W2, b2,
           att0, att1, fuse_W, fuse_b, fuse_att):
    nei0f = jnp.reshape(nei0.astype(jnp.int32), (NEIROWS, IDXW))
    nei1f = jnp.reshape(nei1.astype(jnp.int32), (NEIROWS, IDXW))

    h1, h2, t0, s1, t1, s2 = _run_proj(
        feat0, feat1, feat2, W0, b0, W1, b1, W2, b2, att0, att1)

    vs, vt = _run_sc(h1, h2, t0, s1, t1, s2, nei0f, nei1f)

    et, es, beta = _run_fuse(vt, vs, fuse_W, fuse_b, fuse_att)

    return _run_combine(beta, et, es)


# speedup vs baseline: 10.5619x; 1.0285x over previous
import functools

import jax
import jax.numpy as jnp
from jax import lax
from jax.experimental import pallas as pl
from jax.experimental.pallas import tpu as pltpu
from jax.experimental.pallas import tpu_sc as plsc

N = 10000
D = 128
H = 64
S = 32

RB = 2000
GRID = N // RB
WOUT = 128

C = 16
BROWS = C * S
NCHUNK = N // C
IDXW = 128
GPC = BROWS // IDXW
NEIROWS = N * S // IDXW


def _elu(x):
    return jnp.where(x > 0, x, jnp.exp(x) - 1.0)


def _score(h, avec):
    return jnp.sum(h * avec[None, :], axis=1).reshape(1, 1, RB)


def _proj_a_body(f0_r, f1_r, w0_r, b0_r, w1_r, b1_r, a0_r, a1_r,
                 h1_o, t0_o, s1_o, t1_o):
    def proj(f_r, w_r, b_r):
        h = lax.dot_general(f_r[...], w_r[...], (((1,), (1,)), ((), ())),
                            preferred_element_type=jnp.float32) + b_r[...]
        return _elu(h)

    h0 = proj(f0_r, w0_r, b0_r)
    h1 = proj(f1_r, w1_r, b1_r)
    h1_o[...] = h1
    att0 = a0_r[...]
    att1 = a1_r[...]
    t0_o[...] = _score(h0, att0[0, :H])
    s1_o[...] = _score(h1, att0[0, H:])
    t1_o[...] = _score(h0, att1[0, :H])


def _proj_b_body(f2_r, w2_r, b2_r, a1_r, h2_o, s2_o):
    h = lax.dot_general(f2_r[...], w2_r[...], (((1,), (1,)), ((), ())),
                        preferred_element_type=jnp.float32) + b2_r[...]
    h2 = _elu(h)
    h2_o[...] = h2
    s2_o[...] = _score(h2, a1_r[...][0, H:])


_ROW_D = pl.BlockSpec((RB, D), lambda i: (i, 0))
_ROW_H = pl.BlockSpec((RB, H), lambda i: (i, 0))
_SCORE = pl.BlockSpec((1, 1, RB), lambda i: (i, 0, 0))


def _full(shape):
    return pl.BlockSpec(shape, lambda i: tuple(0 for _ in shape))


def _run_proj_a(feat0, feat1, W0, b0, W1, b1, att0, att1):
    return pl.pallas_call(
        _proj_a_body,
        grid=(GRID,),
        in_specs=[_ROW_D, _ROW_D,
                  _full((H, D)), _full((1, H)),
                  _full((H, D)), _full((1, H)),
                  _full((1, 2 * H)), _full((1, 2 * H))],
        out_specs=[_ROW_H, _SCORE, _SCORE, _SCORE],
        out_shape=[jax.ShapeDtypeStruct((N, H), jnp.float32)] +
                  [jax.ShapeDtypeStruct((GRID, 1, RB), jnp.float32)] * 3,
    )(feat0, feat1, W0, b0.reshape(1, H), W1, b1.reshape(1, H), att0, att1)


def _run_proj_b(feat2, W2, b2, att1):
    return pl.pallas_call(
        _proj_b_body,
        grid=(GRID,),
        in_specs=[_ROW_D, _full((H, D)), _full((1, H)), _full((1, 2 * H))],
        out_specs=[_ROW_H, _SCORE],
        out_shape=[jax.ShapeDtypeStruct((N, H), jnp.float32),
                   jax.ShapeDtypeStruct((GRID, 1, RB), jnp.float32)],
    )(feat2, W2, b2.reshape(1, H), att1)


NCMAX = 20


def _sc_body(h_hbm, t_hbm, s_hbm, nei_hbm,
             out_hbm,
             tgt_v, src_v, idx_v, rows0_v, rows1_v, w_v, out0_v, out1_v,
             rsem0, rsem1, osem0, osem1):
    info = plsc.get_sparse_core_info()
    nw = info.num_cores * info.num_subcores
    wid = lax.axis_index("s") * info.num_cores + lax.axis_index("c")
    start = (wid * NCHUNK) // nw
    end = ((wid + 1) * NCHUNK) // nw
    nc = end - start

    pltpu.sync_copy(t_hbm, tgt_v)
    pltpu.sync_copy(s_hbm, src_v)
    pltpu.sync_copy(nei_hbm.at[pl.ds(start * GPC, NCMAX * GPC)], idx_v)

    def issue_gathers(j, rows_b, sem):
        for g in range(GPC):
            pltpu.async_copy(h_hbm.at[idx_v.at[j * GPC + g]],
                             rows_b.at[pl.ds(g * IDXW, IDXW)], sem)

    def wait_gathers(j, rows_b, sem):
        for g in range(GPC):
            pltpu.make_async_copy(h_hbm.at[idx_v.at[j * GPC + g]],
                                  rows_b.at[pl.ds(g * IDXW, IDXW)],
                                  sem).wait()

    def process(j, j2, rows_b, sem, out_b, osem):
        cid = start + j
        node0 = cid * C
        tgt_chunk = tgt_v[pl.ds(node0, C)]

        for n in range(C):
            r = j * GPC + n // (IDXW // S)
            col = (n % (IDXW // S)) * S
            iv0 = idx_v[r, pl.ds(col, 16)]
            iv1 = idx_v[r, pl.ds(col + 16, 16)]
            sv0 = plsc.load_gather(src_v, [iv0])
            sv1 = plsc.load_gather(src_v, [iv1])
            t = tgt_chunk[n]
            a0 = t + sv0
            a0 = jnp.where(a0 > 0, a0, a0 * 0.01)
            a1 = t + sv1
            a1 = jnp.where(a1 > 0, a1, a1 * 0.01)
            m = jnp.maximum(jnp.max(a0), jnp.max(a1))
            e0 = jnp.exp(a0 - m)
            e1 = jnp.exp(a1 - m)
            dv = jnp.full((16,), jnp.sum(e0) + jnp.sum(e1), jnp.float32)
            rv = 1.0 / dv
            w_v[pl.ds(n * S, 16)] = e0 * rv
            w_v[pl.ds(n * S + 16, 16)] = e1 * rv

        @pl.when(j2 > 0)
        def _():
            pltpu.make_async_copy(
                out_b, out_hbm.at[pl.ds((cid - 2) * C, C)], osem).wait()

        wait_gathers(j, rows_b, sem)

        def abody(n, _):
            base = n * S
            wv0 = w_v[pl.ds(base, 16)]
            wv1 = w_v[pl.ds(base + 16, 16)]
            accs = [jnp.zeros((16,), jnp.float32) for _ in range(H // 16)]
            for s in range(S):
                w = wv0[s] if s < 16 else wv1[s - 16]
                for k in range(H // 16):
                    accs[k] = accs[k] + w * rows_b[base + s, pl.ds(k * 16, 16)]
            for k in range(H // 16):
                out_b[n, pl.ds(k * 16, 16)] = accs[k]
            return 0

        lax.fori_loop(0, C, abody, 0)
        pltpu.async_copy(out_b, out_hbm.at[pl.ds(node0, C)], osem)

        @pl.when(j + 2 < nc)
        def _():
            issue_gathers(j + 2, rows_b, sem)

    issue_gathers(0, rows0_v, rsem0)
    issue_gathers(1, rows1_v, rsem1)

    def pair_body(j2, _):
        process(2 * j2, j2, rows0_v, rsem0, out0_v, osem0)

        @pl.when(2 * j2 + 1 < nc)
        def _():
            process(2 * j2 + 1, j2, rows1_v, rsem1, out1_v, osem1)

        return 0

    lax.fori_loop(0, (nc + 1) // 2, pair_body, 0)

    je = ((nc - 1) // 2) * 2
    jo = ((nc - 2) // 2) * 2 + 1
    pltpu.make_async_copy(
        out0_v, out_hbm.at[pl.ds((start + je) * C, C)], osem0).wait()
    pltpu.make_async_copy(
        out1_v, out_hbm.at[pl.ds((start + jo) * C, C)], osem1).wait()


def _run_sc(h, t, s, neif):
    mesh = plsc.VectorSubcoreMesh(core_axis_name="c", subcore_axis_name="s")
    fn = pl.kernel(
        _sc_body,
        mesh=mesh,
        compiler_params=pltpu.CompilerParams(needs_layout_passes=False,
                                             use_tc_tiling_on_sc=False),
        out_type=jax.ShapeDtypeStruct((N, WOUT), jnp.float32),
        scratch_types=[
            pltpu.VMEM((N,), jnp.float32),
            pltpu.VMEM((N,), jnp.float32),
            pltpu.VMEM((NCMAX * GPC, IDXW), jnp.int32),
            pltpu.VMEM((BROWS, H), jnp.float32),
            pltpu.VMEM((BROWS, H), jnp.float32),
            pltpu.VMEM((BROWS,), jnp.float32),
            pltpu.VMEM((C, WOUT), jnp.float32),
            pltpu.VMEM((C, WOUT), jnp.float32),
            pltpu.SemaphoreType.DMA,
            pltpu.SemaphoreType.DMA,
            pltpu.SemaphoreType.DMA,
            pltpu.SemaphoreType.DMA,
        ],
    )
    return fn(h, t, s, neif)


def _fuse_s_body(vs_r, fw_r, fb_r, es_o, ssum_o, sums):
    i = pl.program_id(0)
    es = _elu(vs_r[:, :H])
    es_o[...] = es
    part = jnp.sum(
        jnp.tanh(lax.dot_general(es, fw_r[...], (((1,), (1,)), ((), ())),
                                 preferred_element_type=jnp.float32)
                 + fb_r[...]), axis=0, keepdims=True)
    prev = jnp.where(i == 0, jnp.zeros_like(part), sums[...])
    sums[...] = prev + part

    @pl.when(i == GRID - 1)
    def _():
        ssum_o[...] = sums[...]


def _run_fuse_s(vs, fuse_W, fuse_b):
    roww = pl.BlockSpec((RB, WOUT), lambda i: (i, 0))
    return pl.pallas_call(
        _fuse_s_body,
        grid=(GRID,),
        in_specs=[roww, _full((H, H)), _full((1, H))],
        out_specs=[_ROW_H, pl.BlockSpec((1, H), lambda i: (0, 0))],
        out_shape=[jax.ShapeDtypeStruct((N, H), jnp.float32),
                   jax.ShapeDtypeStruct((1, H), jnp.float32)],
        scratch_shapes=[pltpu.VMEM((1, H), jnp.float32)],
    )(vs, fuse_W, fuse_b.reshape(1, H))


def _fuse_t_body(vt_r, fw_r, fb_r, fa_r, ssum_r, et_o, beta_o, sums):
    i = pl.program_id(0)
    et = _elu(vt_r[:, :H])
    et_o[...] = et
    part = jnp.sum(
        jnp.tanh(lax.dot_general(et, fw_r[...], (((1,), (1,)), ((), ())),
                                 preferred_element_type=jnp.float32)
                 + fb_r[...]), axis=0, keepdims=True)
    prev = jnp.where(i == 0, jnp.zeros_like(part), sums[...])
    sums[...] = prev + part

    @pl.when(i == GRID - 1)
    def _():
        sp = jnp.concatenate([sums[...], ssum_r[...]], axis=0) * (1.0 / N)
        b2 = jnp.sum(sp * fa_r[...], axis=1, keepdims=True)
        e = jnp.exp(b2 - jnp.max(b2))
        beta_o[...] = e / jnp.sum(e)


def _run_fuse_t(vt, fuse_W, fuse_b, fuse_att, ssum):
    roww = pl.BlockSpec((RB, WOUT), lambda i: (i, 0))
    return pl.pallas_call(
        _fuse_t_body,
        grid=(GRID,),
        in_specs=[roww, _full((H, H)), _full((1, H)), _full((1, H)),
                  _full((1, H))],
        out_specs=[_ROW_H, pl.BlockSpec((2, 1), lambda i: (0, 0))],
        out_shape=[jax.ShapeDtypeStruct((N, H), jnp.float32),
                   jax.ShapeDtypeStruct((2, 1), jnp.float32)],
        scratch_shapes=[pltpu.VMEM((1, H), jnp.float32)],
    )(vt, fuse_W, fuse_b.reshape(1, H), fuse_att, ssum)


def _comb_body(b_r, et_r, es_r, z_o):
    z_o[...] = b_r[0] * et_r[...] + b_r[1] * es_r[...]


def _run_combine(beta, et, es):
    return pl.pallas_call(
        _comb_body,
        grid=(GRID,),
        in_specs=[pl.BlockSpec(memory_space=pltpu.SMEM), _ROW_H, _ROW_H],
        out_specs=_ROW_H,
        out_shape=jax.ShapeDtypeStruct((N, H), jnp.float32),
    )(beta.reshape(2), et, es)


@jax.jit
def kernel(feat0, feat1, feat2, nei0, nei1, W0, b0, W1, b1, W2, b2,
           att0, att1, fuse_W, fuse_b, fuse_att):
    nei0f = jnp.reshape(nei0.astype(jnp.int32), (NEIROWS, IDXW))
    nei1f = jnp.reshape(nei1.astype(jnp.int32), (NEIROWS, IDXW))

    h1, t0, s1, t1 = _run_proj_a(feat0, feat1, W0, b0, W1, b1, att0, att1)
    flat = lambda x: x.reshape(N)

    vs = _run_sc(h1, flat(t0), flat(s1), nei0f)

    h2, s2 = _run_proj_b(feat2, W2, b2, att1)
    vt = _run_sc(h2, flat(t1), flat(s2), nei1f)

    es, ssum = _run_fuse_s(vs, fuse_W, fuse_b)
    et, beta = _run_fuse_t(vt, fuse_W, fuse_b, fuse_att, ssum)

    return _run_combine(beta, et, es)


# speedup vs baseline: 10.7232x; 1.0153x over previous
import functools

import jax
import jax.numpy as jnp
from jax import lax
from jax.experimental import pallas as pl
from jax.experimental.pallas import tpu as pltpu
from jax.experimental.pallas import tpu_sc as plsc

N = 10000
D = 128
H = 64
S = 32

RB = 2000
GRID = N // RB
WOUT = 128

C = 16
BROWS = C * S
NCHUNK = N // C
IDXW = 128
GPC = BROWS // IDXW
NEIROWS = N * S // IDXW


def _elu(x):
    return jnp.where(x > 0, x, jnp.exp(x) - 1.0)


def _score(h, avec):
    return jnp.sum(h * avec[None, :], axis=1).reshape(1, 1, RB)


def _proj_a_body(f0_r, f1_r, w0_r, b0_r, w1_r, b1_r, a0_r, a1_r,
                 h1_o, t0_o, s1_o, t1_o):
    def proj(f_r, w_r, b_r):
        h = lax.dot_general(f_r[...], w_r[...], (((1,), (1,)), ((), ())),
                            preferred_element_type=jnp.float32) + b_r[...]
        return _elu(h)

    h0 = proj(f0_r, w0_r, b0_r)
    h1 = proj(f1_r, w1_r, b1_r)
    h1_o[...] = h1
    att0 = a0_r[...]
    att1 = a1_r[...]
    t0_o[...] = _score(h0, att0[0, :H])
    s1_o[...] = _score(h1, att0[0, H:])
    t1_o[...] = _score(h0, att1[0, :H])


def _proj_b_body(f2_r, w2_r, b2_r, a1_r, h2_o, s2_o):
    h = lax.dot_general(f2_r[...], w2_r[...], (((1,), (1,)), ((), ())),
                        preferred_element_type=jnp.float32) + b2_r[...]
    h2 = _elu(h)
    h2_o[...] = h2
    s2_o[...] = _score(h2, a1_r[...][0, H:])


_ROW_D = pl.BlockSpec((RB, D), lambda i: (i, 0))
_ROW_H = pl.BlockSpec((RB, H), lambda i: (i, 0))
_ROW_W = pl.BlockSpec((RB, WOUT), lambda i: (i, 0))
_SCORE = pl.BlockSpec((1, 1, RB), lambda i: (i, 0, 0))


def _full(shape):
    return pl.BlockSpec(shape, lambda i: tuple(0 for _ in shape))


def _run_proj_a(feat0, feat1, W0, b0, W1, b1, att0, att1):
    return pl.pallas_call(
        _proj_a_body,
        grid=(GRID,),
        in_specs=[_ROW_D, _ROW_D,
                  _full((H, D)), _full((1, H)),
                  _full((H, D)), _full((1, H)),
                  _full((1, 2 * H)), _full((1, 2 * H))],
        out_specs=[_ROW_H, _SCORE, _SCORE, _SCORE],
        out_shape=[jax.ShapeDtypeStruct((N, H), jnp.float32)] +
                  [jax.ShapeDtypeStruct((GRID, 1, RB), jnp.float32)] * 3,
    )(feat0, feat1, W0, b0.reshape(1, H), W1, b1.reshape(1, H), att0, att1)


def _run_proj_b(feat2, W2, b2, att1):
    return pl.pallas_call(
        _proj_b_body,
        grid=(GRID,),
        in_specs=[_ROW_D, _full((H, D)), _full((1, H)), _full((1, 2 * H))],
        out_specs=[_ROW_H, _SCORE],
        out_shape=[jax.ShapeDtypeStruct((N, H), jnp.float32),
                   jax.ShapeDtypeStruct((GRID, 1, RB), jnp.float32)],
    )(feat2, W2, b2.reshape(1, H), att1)


NCMAX = 20


def _sc_body(h_hbm, t_hbm, s_hbm, nei_hbm,
             out_hbm,
             tgt_v, src_v, idx_v, rows0_v, rows1_v, w_v, out0_v, out1_v,
             rsem0, rsem1, osem0, osem1):
    info = plsc.get_sparse_core_info()
    nw = info.num_cores * info.num_subcores
    wid = lax.axis_index("s") * info.num_cores + lax.axis_index("c")
    start = (wid * NCHUNK) // nw
    end = ((wid + 1) * NCHUNK) // nw
    nc = end - start

    pltpu.sync_copy(t_hbm, tgt_v)
    pltpu.sync_copy(s_hbm, src_v)
    pltpu.sync_copy(nei_hbm.at[pl.ds(start * GPC, NCMAX * GPC)], idx_v)

    def issue_gathers(j, rows_b, sem):
        for g in range(GPC):
            pltpu.async_copy(h_hbm.at[idx_v.at[j * GPC + g]],
                             rows_b.at[pl.ds(g * IDXW, IDXW)], sem)

    def wait_gathers(j, rows_b, sem):
        for g in range(GPC):
            pltpu.make_async_copy(h_hbm.at[idx_v.at[j * GPC + g]],
                                  rows_b.at[pl.ds(g * IDXW, IDXW)],
                                  sem).wait()

    def process(j, j2, rows_b, sem, out_b, osem):
        cid = start + j
        node0 = cid * C
        tgt_chunk = tgt_v[pl.ds(node0, C)]

        for n in range(C):
            r = j * GPC + n // (IDXW // S)
            col = (n % (IDXW // S)) * S
            iv0 = idx_v[r, pl.ds(col, 16)]
            iv1 = idx_v[r, pl.ds(col + 16, 16)]
            sv0 = plsc.load_gather(src_v, [iv0])
            sv1 = plsc.load_gather(src_v, [iv1])
            t = tgt_chunk[n]
            a0 = t + sv0
            a0 = jnp.where(a0 > 0, a0, a0 * 0.01)
            a1 = t + sv1
            a1 = jnp.where(a1 > 0, a1, a1 * 0.01)
            m = jnp.maximum(jnp.max(a0), jnp.max(a1))
            e0 = jnp.exp(a0 - m)
            e1 = jnp.exp(a1 - m)
            dv = jnp.full((16,), jnp.sum(e0) + jnp.sum(e1), jnp.float32)
            rv = 1.0 / dv
            w_v[pl.ds(n * S, 16)] = e0 * rv
            w_v[pl.ds(n * S + 16, 16)] = e1 * rv

        @pl.when(j2 > 0)
        def _():
            pltpu.make_async_copy(
                out_b, out_hbm.at[pl.ds((cid - 2) * C, C)], osem).wait()

        wait_gathers(j, rows_b, sem)

        def abody(n, _):
            base = n * S
            wv0 = w_v[pl.ds(base, 16)]
            wv1 = w_v[pl.ds(base + 16, 16)]
            accs = [jnp.zeros((16,), jnp.float32) for _ in range(H // 16)]
            for s in range(S):
                w = wv0[s] if s < 16 else wv1[s - 16]
                for k in range(H // 16):
                    accs[k] = accs[k] + w * rows_b[base + s, pl.ds(k * 16, 16)]
            for k in range(H // 16):
                out_b[n, pl.ds(k * 16, 16)] = accs[k]
            return 0

        lax.fori_loop(0, C, abody, 0)
        pltpu.async_copy(out_b, out_hbm.at[pl.ds(node0, C)], osem)

        @pl.when(j + 2 < nc)
        def _():
            issue_gathers(j + 2, rows_b, sem)

    issue_gathers(0, rows0_v, rsem0)
    issue_gathers(1, rows1_v, rsem1)

    def pair_body(j2, _):
        process(2 * j2, j2, rows0_v, rsem0, out0_v, osem0)

        @pl.when(2 * j2 + 1 < nc)
        def _():
            process(2 * j2 + 1, j2, rows1_v, rsem1, out1_v, osem1)

        return 0

    lax.fori_loop(0, (nc + 1) // 2, pair_body, 0)

    je = ((nc - 1) // 2) * 2
    jo = ((nc - 2) // 2) * 2 + 1
    pltpu.make_async_copy(
        out0_v, out_hbm.at[pl.ds((start + je) * C, C)], osem0).wait()
    pltpu.make_async_copy(
        out1_v, out_hbm.at[pl.ds((start + jo) * C, C)], osem1).wait()


def _run_sc(h, t, s, neif):
    mesh = plsc.VectorSubcoreMesh(core_axis_name="c", subcore_axis_name="s")
    fn = pl.kernel(
        _sc_body,
        mesh=mesh,
        compiler_params=pltpu.CompilerParams(needs_layout_passes=False,
                                             use_tc_tiling_on_sc=False),
        out_type=jax.ShapeDtypeStruct((N, WOUT), jnp.float32),
        scratch_types=[
            pltpu.VMEM((N,), jnp.float32),
            pltpu.VMEM((N,), jnp.float32),
            pltpu.VMEM((NCMAX * GPC, IDXW), jnp.int32),
            pltpu.VMEM((BROWS, H), jnp.float32),
            pltpu.VMEM((BROWS, H), jnp.float32),
            pltpu.VMEM((BROWS,), jnp.float32),
            pltpu.VMEM((C, WOUT), jnp.float32),
            pltpu.VMEM((C, WOUT), jnp.float32),
            pltpu.SemaphoreType.DMA,
            pltpu.SemaphoreType.DMA,
            pltpu.SemaphoreType.DMA,
            pltpu.SemaphoreType.DMA,
        ],
    )
    return fn(h, t, s, neif)


def _fuse_s_body(vs_r, fw_r, fb_r, es_o, ssum_o, sums):
    i = pl.program_id(0)
    es = _elu(vs_r[:, :H])
    es_o[...] = es
    part = jnp.sum(
        jnp.tanh(lax.dot_general(es, fw_r[...], (((1,), (1,)), ((), ())),
                                 preferred_element_type=jnp.float32)
                 + fb_r[...]), axis=0, keepdims=True)
    prev = jnp.where(i == 0, jnp.zeros_like(part), sums[...])
    sums[...] = prev + part

    @pl.when(i == GRID - 1)
    def _():
        ssum_o[...] = sums[...]


def _run_fuse_s(vs, fuse_W, fuse_b):
    roww = pl.BlockSpec((RB, WOUT), lambda i: (i, 0))
    return pl.pallas_call(
        _fuse_s_body,
        grid=(GRID,),
        in_specs=[roww, _full((H, H)), _full((1, H))],
        out_specs=[_ROW_H, pl.BlockSpec((1, H), lambda i: (0, 0))],
        out_shape=[jax.ShapeDtypeStruct((N, H), jnp.float32),
                   jax.ShapeDtypeStruct((1, H), jnp.float32)],
        scratch_shapes=[pltpu.VMEM((1, H), jnp.float32)],
    )(vs, fuse_W, fuse_b.reshape(1, H))


def _fuse_t_body(vt_r, es_r, fw_r, fb_r, fa_r, ssum_r, z_o,
                 sums, beta_s, et_s):
    i = pl.program_id(0)

    @pl.when(i < GRID)
    def _():
        et = _elu(vt_r[:, :H])
        et_s[pl.ds(i * RB, RB)] = et
        z_o[...] = et
        part = jnp.sum(
            jnp.tanh(lax.dot_general(et, fw_r[...], (((1,), (1,)), ((), ())),
                                     preferred_element_type=jnp.float32)
                     + fb_r[...]), axis=0, keepdims=True)
        prev = jnp.where(i == 0, jnp.zeros_like(part), sums[...])
        sums[...] = prev + part

        @pl.when(i == GRID - 1)
        def _():
            sp = jnp.concatenate([sums[...], ssum_r[...]], axis=0) * (1.0 / N)
            b2 = jnp.sum(sp * fa_r[...], axis=1, keepdims=True)
            e = jnp.exp(b2 - jnp.max(b2))
            beta_s[...] = e / jnp.sum(e)

    @pl.when(i >= GRID)
    def _():
        j = i - GRID
        et = et_s[pl.ds(j * RB, RB)]
        z_o[...] = beta_s[0, 0] * et + beta_s[1, 0] * es_r[...]


def _run_fuse_t(vt, es, fuse_W, fuse_b, fuse_att, ssum):
    roww = pl.BlockSpec((RB, WOUT),
                        lambda i: (jnp.minimum(i, GRID - 1), 0))
    esrow = pl.BlockSpec((RB, H),
                         lambda i: (jnp.maximum(i - GRID, 0), 0))
    zrow = pl.BlockSpec((RB, H), lambda i: (i % GRID, 0))
    return pl.pallas_call(
        _fuse_t_body,
        grid=(2 * GRID,),
        in_specs=[roww, esrow, _full((H, H)), _full((1, H)), _full((1, H)),
                  _full((1, H))],
        out_specs=zrow,
        out_shape=jax.ShapeDtypeStruct((N, H), jnp.float32),
        scratch_shapes=[pltpu.VMEM((1, H), jnp.float32),
                        pltpu.VMEM((2, 1), jnp.float32),
                        pltpu.VMEM((N, H), jnp.float32)],
    )(vt, es, fuse_W, fuse_b.reshape(1, H), fuse_att, ssum)


@jax.jit
def kernel(feat0, feat1, feat2, nei0, nei1, W0, b0, W1, b1, W2, b2,
           att0, att1, fuse_W, fuse_b, fuse_att):
    nei0f = jnp.reshape(nei0.astype(jnp.int32), (NEIROWS, IDXW))
    nei1f = jnp.reshape(nei1.astype(jnp.int32), (NEIROWS, IDXW))

    h1, t0, s1, t1 = _run_proj_a(feat0, feat1, W0, b0, W1, b1, att0, att1)
    flat = lambda x: x.reshape(N)

    vs = _run_sc(h1, flat(t0), flat(s1), nei0f)

    h2, s2 = _run_proj_b(feat2, W2, b2, att1)
    vt = _run_sc(h2, flat(t1), flat(s2), nei1f)

    es, ssum = _run_fuse_s(vs, fuse_W, fuse_b)
    return _run_fuse_t(vt, es, fuse_W, fuse_b, fuse_att, ssum)


# speedup vs baseline: 11.1279x; 1.0377x over previous
import functools

import jax
import jax.numpy as jnp
from jax import lax
from jax.experimental import pallas as pl
from jax.experimental.pallas import tpu as pltpu
from jax.experimental.pallas import tpu_sc as plsc

N = 10000
D = 128
H = 64
S = 32

RB = 2000
GRID = N // RB
WOUT = 128

C = 16
BROWS = C * S
NCHUNK = N // C
IDXW = 128
GPC = BROWS // IDXW
NEIROWS = N * S // IDXW


def _elu(x):
    return jnp.where(x > 0, x, jnp.exp(x) - 1.0)


def _score(h, avec):
    return jnp.sum(h * avec[None, :], axis=1).reshape(1, 1, RB)


def _proj_a_body(f0_r, f1_r, w0_r, b0_r, w1_r, b1_r, a0_r, a1_r,
                 h1_o, t0_o, s1_o, t1_o):
    def proj(f_r, w_r, b_r):
        h = lax.dot_general(f_r[...], w_r[...], (((1,), (1,)), ((), ())),
                            preferred_element_type=jnp.float32) + b_r[...]
        return _elu(h)

    h0 = proj(f0_r, w0_r, b0_r)
    h1 = proj(f1_r, w1_r, b1_r)
    h1_o[...] = h1
    att0 = a0_r[...]
    att1 = a1_r[...]
    t0_o[...] = _score(h0, att0[0, :H])
    s1_o[...] = _score(h1, att0[0, H:])
    t1_o[...] = _score(h0, att1[0, :H])


def _proj_b_body(f2_r, w2_r, b2_r, a1_r, h2_o, s2_o):
    h = lax.dot_general(f2_r[...], w2_r[...], (((1,), (1,)), ((), ())),
                        preferred_element_type=jnp.float32) + b2_r[...]
    h2 = _elu(h)
    h2_o[...] = h2
    s2_o[...] = _score(h2, a1_r[...][0, H:])


_ROW_D = pl.BlockSpec((RB, D), lambda i: (i, 0))
_ROW_H = pl.BlockSpec((RB, H), lambda i: (i, 0))
_ROW_W = pl.BlockSpec((RB, WOUT), lambda i: (i, 0))
_SCORE = pl.BlockSpec((1, 1, RB), lambda i: (i, 0, 0))


def _full(shape):
    return pl.BlockSpec(shape, lambda i: tuple(0 for _ in shape))


def _run_proj_a(feat0, feat1, W0, b0, W1, b1, att0, att1):
    return pl.pallas_call(
        _proj_a_body,
        grid=(GRID,),
        in_specs=[_ROW_D, _ROW_D,
                  _full((H, D)), _full((1, H)),
                  _full((H, D)), _full((1, H)),
                  _full((1, 2 * H)), _full((1, 2 * H))],
        out_specs=[_ROW_H, _SCORE, _SCORE, _SCORE],
        out_shape=[jax.ShapeDtypeStruct((N, H), jnp.float32)] +
                  [jax.ShapeDtypeStruct((GRID, 1, RB), jnp.float32)] * 3,
    )(feat0, feat1, W0, b0.reshape(1, H), W1, b1.reshape(1, H), att0, att1)


def _run_proj_b(feat2, W2, b2, att1):
    return pl.pallas_call(
        _proj_b_body,
        grid=(GRID,),
        in_specs=[_ROW_D, _full((H, D)), _full((1, H)), _full((1, 2 * H))],
        out_specs=[_ROW_H, _SCORE],
        out_shape=[jax.ShapeDtypeStruct((N, H), jnp.float32),
                   jax.ShapeDtypeStruct((GRID, 1, RB), jnp.float32)],
    )(feat2, W2, b2.reshape(1, H), att1)


NCMAX = 20


def _sc_body(h_hbm, t_hbm, s_hbm, nei_hbm,
             out_hbm,
             tgt_v, src_v, idx_v, rows0_v, rows1_v, w_v, out0_v, out1_v,
             rsem0, rsem1, osem0, osem1, ssem):
    info = plsc.get_sparse_core_info()
    nw = info.num_cores * info.num_subcores
    wid = lax.axis_index("s") * info.num_cores + lax.axis_index("c")
    start = (wid * NCHUNK) // nw
    end = ((wid + 1) * NCHUNK) // nw
    nc = end - start

    pltpu.sync_copy(nei_hbm.at[pl.ds(start * GPC, NCMAX * GPC)], idx_v)
    pltpu.async_copy(t_hbm.at[pl.ds(start * C, NCMAX * C)], tgt_v, ssem)
    pltpu.async_copy(s_hbm, src_v, ssem)

    def issue_gathers(j, rows_b, sem):
        for g in range(GPC):
            pltpu.async_copy(h_hbm.at[idx_v.at[j * GPC + g]],
                             rows_b.at[pl.ds(g * IDXW, IDXW)], sem)

    def wait_gathers(j, rows_b, sem):
        for g in range(GPC):
            pltpu.make_async_copy(h_hbm.at[idx_v.at[j * GPC + g]],
                                  rows_b.at[pl.ds(g * IDXW, IDXW)],
                                  sem).wait()

    def process(j, j2, rows_b, sem, out_b, osem):
        cid = start + j
        node0 = cid * C
        tgt_chunk = tgt_v[pl.ds(j * C, C)]

        for n in range(C):
            r = j * GPC + n // (IDXW // S)
            col = (n % (IDXW // S)) * S
            iv0 = idx_v[r, pl.ds(col, 16)]
            iv1 = idx_v[r, pl.ds(col + 16, 16)]
            sv0 = plsc.load_gather(src_v, [iv0])
            sv1 = plsc.load_gather(src_v, [iv1])
            t = tgt_chunk[n]
            a0 = t + sv0
            a0 = jnp.where(a0 > 0, a0, a0 * 0.01)
            a1 = t + sv1
            a1 = jnp.where(a1 > 0, a1, a1 * 0.01)
            m = jnp.maximum(jnp.max(a0), jnp.max(a1))
            e0 = jnp.exp(a0 - m)
            e1 = jnp.exp(a1 - m)
            dv = jnp.full((16,), jnp.sum(e0) + jnp.sum(e1), jnp.float32)
            rv = 1.0 / dv
            w_v[pl.ds(n * S, 16)] = e0 * rv
            w_v[pl.ds(n * S + 16, 16)] = e1 * rv

        @pl.when(j2 > 0)
        def _():
            pltpu.make_async_copy(
                out_b, out_hbm.at[pl.ds((cid - 2) * C, C)], osem).wait()

        wait_gathers(j, rows_b, sem)

        def abody(n, _):
            base = n * S
            wv0 = w_v[pl.ds(base, 16)]
            wv1 = w_v[pl.ds(base + 16, 16)]
            accs = [jnp.zeros((16,), jnp.float32) for _ in range(H // 16)]
            for s in range(S):
                w = wv0[s] if s < 16 else wv1[s - 16]
                for k in range(H // 16):
                    accs[k] = accs[k] + w * rows_b[base + s, pl.ds(k * 16, 16)]
            for k in range(H // 16):
                out_b[n, pl.ds(k * 16, 16)] = accs[k]
            return 0

        lax.fori_loop(0, C, abody, 0)
        pltpu.async_copy(out_b, out_hbm.at[pl.ds(node0, C)], osem)

        @pl.when(j + 2 < nc)
        def _():
            issue_gathers(j + 2, rows_b, sem)

    issue_gathers(0, rows0_v, rsem0)
    issue_gathers(1, rows1_v, rsem1)
    pltpu.make_async_copy(
        t_hbm.at[pl.ds(start * C, NCMAX * C)], tgt_v, ssem).wait()
    pltpu.make_async_copy(s_hbm, src_v, ssem).wait()

    def pair_body(j2, _):
        process(2 * j2, j2, rows0_v, rsem0, out0_v, osem0)

        @pl.when(2 * j2 + 1 < nc)
        def _():
            process(2 * j2 + 1, j2, rows1_v, rsem1, out1_v, osem1)

        return 0

    lax.fori_loop(0, (nc + 1) // 2, pair_body, 0)

    je = ((nc - 1) // 2) * 2
    jo = ((nc - 2) // 2) * 2 + 1
    pltpu.make_async_copy(
        out0_v, out_hbm.at[pl.ds((start + je) * C, C)], osem0).wait()
    pltpu.make_async_copy(
        out1_v, out_hbm.at[pl.ds((start + jo) * C, C)], osem1).wait()


def _run_sc(h, t, s, neif):
    mesh = plsc.VectorSubcoreMesh(core_axis_name="c", subcore_axis_name="s")
    fn = pl.kernel(
        _sc_body,
        mesh=mesh,
        compiler_params=pltpu.CompilerParams(needs_layout_passes=False,
                                             use_tc_tiling_on_sc=False),
        out_type=jax.ShapeDtypeStruct((N, WOUT), jnp.float32),
        scratch_types=[
            pltpu.VMEM((NCMAX * C,), jnp.float32),
            pltpu.VMEM((N,), jnp.float32),
            pltpu.VMEM((NCMAX * GPC, IDXW), jnp.int32),
            pltpu.VMEM((BROWS, H), jnp.float32),
            pltpu.VMEM((BROWS, H), jnp.float32),
            pltpu.VMEM((BROWS,), jnp.float32),
            pltpu.VMEM((C, WOUT), jnp.float32),
            pltpu.VMEM((C, WOUT), jnp.float32),
            pltpu.SemaphoreType.DMA,
            pltpu.SemaphoreType.DMA,
            pltpu.SemaphoreType.DMA,
            pltpu.SemaphoreType.DMA,
            pltpu.SemaphoreType.DMA,
        ],
    )
    return fn(h, t, s, neif)


def _fuse_s_body(vs_r, fw_r, fb_r, es_o, ssum_o, sums):
    i = pl.program_id(0)
    es = _elu(vs_r[:, :H])
    es_o[...] = es
    part = jnp.sum(
        jnp.tanh(lax.dot_general(es, fw_r[...], (((1,), (1,)), ((), ())),
                                 preferred_element_type=jnp.float32)
                 + fb_r[...]), axis=0, keepdims=True)
    prev = jnp.where(i == 0, jnp.zeros_like(part), sums[...])
    sums[...] = prev + part

    @pl.when(i == GRID - 1)
    def _():
        ssum_o[...] = sums[...]


def _run_fuse_s(vs, fuse_W, fuse_b):
    roww = pl.BlockSpec((RB, WOUT), lambda i: (i, 0))
    return pl.pallas_call(
        _fuse_s_body,
        grid=(GRID,),
        in_specs=[roww, _full((H, H)), _full((1, H))],
        out_specs=[_ROW_H, pl.BlockSpec((1, H), lambda i: (0, 0))],
        out_shape=[jax.ShapeDtypeStruct((N, H), jnp.float32),
                   jax.ShapeDtypeStruct((1, H), jnp.float32)],
        scratch_shapes=[pltpu.VMEM((1, H), jnp.float32)],
    )(vs, fuse_W, fuse_b.reshape(1, H))


def _fuse_t_body(vt_r, es_r, fw_r, fb_r, fa_r, ssum_r, z_o,
                 sums, beta_s, et_s):
    i = pl.program_id(0)

    @pl.when(i < GRID)
    def _():
        et = _elu(vt_r[:, :H])
        et_s[pl.ds(i * RB, RB)] = et
        z_o[...] = et
        part = jnp.sum(
            jnp.tanh(lax.dot_general(et, fw_r[...], (((1,), (1,)), ((), ())),
                                     preferred_element_type=jnp.float32)
                     + fb_r[...]), axis=0, keepdims=True)
        prev = jnp.where(i == 0, jnp.zeros_like(part), sums[...])
        sums[...] = prev + part

        @pl.when(i == GRID - 1)
        def _():
            sp = jnp.concatenate([sums[...], ssum_r[...]], axis=0) * (1.0 / N)
            b2 = jnp.sum(sp * fa_r[...], axis=1, keepdims=True)
            e = jnp.exp(b2 - jnp.max(b2))
            beta_s[...] = e / jnp.sum(e)

    @pl.when(i >= GRID)
    def _():
        j = i - GRID
        et = et_s[pl.ds(j * RB, RB)]
        z_o[...] = beta_s[0, 0] * et + beta_s[1, 0] * es_r[...]


def _run_fuse_t(vt, es, fuse_W, fuse_b, fuse_att, ssum):
    roww = pl.BlockSpec((RB, WOUT),
                        lambda i: (jnp.minimum(i, GRID - 1), 0))
    esrow = pl.BlockSpec((RB, H),
                         lambda i: (jnp.maximum(i - GRID, 0), 0))
    zrow = pl.BlockSpec((RB, H), lambda i: (i % GRID, 0))
    return pl.pallas_call(
        _fuse_t_body,
        grid=(2 * GRID,),
        in_specs=[roww, esrow, _full((H, H)), _full((1, H)), _full((1, H)),
                  _full((1, H))],
        out_specs=zrow,
        out_shape=jax.ShapeDtypeStruct((N, H), jnp.float32),
        scratch_shapes=[pltpu.VMEM((1, H), jnp.float32),
                        pltpu.VMEM((2, 1), jnp.float32),
                        pltpu.VMEM((N, H), jnp.float32)],
    )(vt, es, fuse_W, fuse_b.reshape(1, H), fuse_att, ssum)


@jax.jit
def kernel(feat0, feat1, feat2, nei0, nei1, W0, b0, W1, b1, W2, b2,
           att0, att1, fuse_W, fuse_b, fuse_att):
    nei0f = jnp.reshape(nei0.astype(jnp.int32), (NEIROWS, IDXW))
    nei1f = jnp.reshape(nei1.astype(jnp.int32), (NEIROWS, IDXW))

    h1, t0, s1, t1 = _run_proj_a(feat0, feat1, W0, b0, W1, b1, att0, att1)
    flat = lambda x: x.reshape(N)

    vs = _run_sc(h1, flat(t0), flat(s1), nei0f)

    h2, s2 = _run_proj_b(feat2, W2, b2, att1)
    vt = _run_sc(h2, flat(t1), flat(s2), nei1f)

    es, ssum = _run_fuse_s(vs, fuse_W, fuse_b)
    return _run_fuse_t(vt, es, fuse_W, fuse_b, fuse_att, ssum)


# speedup vs baseline: 11.1345x; 1.0006x over previous
import jax
import jax.numpy as jnp
from jax import lax
from jax.experimental import pallas as pl
from jax.experimental.pallas import tpu as pltpu
from jax.experimental.pallas import tpu_sc as plsc

N = 10000
D = 128
H = 64
S = 32

RB = 2000
GRID = N // RB
WOUT = 128

C = 16
BROWS = C * S
NCHUNK = N // C
IDXW = 128
GPC = BROWS // IDXW
NEIROWS = N * S // IDXW


def _elu(x):
    return jnp.where(x > 0, x, jnp.exp(x) - 1.0)


def _score(h, avec):
    return jnp.sum(h * avec[None, :], axis=1).reshape(1, 1, RB)


def _proj_a_body(f0_r, f1_r, w0_r, b0_r, w1_r, b1_r, a0_r, a1_r,
                 h1_o, t0_o, s1_o, t1_o):
    def proj(f_r, w_r, b_r):
        h = lax.dot_general(f_r[...], w_r[...], (((1,), (1,)), ((), ())),
                            preferred_element_type=jnp.float32) + b_r[...]
        return _elu(h)

    h0 = proj(f0_r, w0_r, b0_r)
    h1 = proj(f1_r, w1_r, b1_r)
    h1_o[...] = h1
    att0 = a0_r[...]
    att1 = a1_r[...]
    t0_o[...] = _score(h0, att0[0, :H])
    s1_o[...] = _score(h1, att0[0, H:])
    t1_o[...] = _score(h0, att1[0, :H])


def _proj_b_body(f2_r, w2_r, b2_r, a1_r, h2_o, s2_o):
    h = lax.dot_general(f2_r[...], w2_r[...], (((1,), (1,)), ((), ())),
                        preferred_element_type=jnp.float32) + b2_r[...]
    h2 = _elu(h)
    h2_o[...] = h2
    s2_o[...] = _score(h2, a1_r[...][0, H:])


_ROW_D = pl.BlockSpec((RB, D), lambda i: (i, 0))
_ROW_H = pl.BlockSpec((RB, H), lambda i: (i, 0))
_SCORE = pl.BlockSpec((1, 1, RB), lambda i: (i, 0, 0))


def _full(shape):
    return pl.BlockSpec(shape, lambda i: tuple(0 for _ in shape))


def _run_proj_a(feat0, feat1, W0, b0, W1, b1, att0, att1):
    return pl.pallas_call(
        _proj_a_body,
        grid=(GRID,),
        in_specs=[_ROW_D, _ROW_D,
                  _full((H, D)), _full((1, H)),
                  _full((H, D)), _full((1, H)),
                  _full((1, 2 * H)), _full((1, 2 * H))],
        out_specs=[_ROW_H, _SCORE, _SCORE, _SCORE],
        out_shape=[jax.ShapeDtypeStruct((N, H), jnp.float32)] +
                  [jax.ShapeDtypeStruct((GRID, 1, RB), jnp.float32)] * 3,
    )(feat0, feat1, W0, b0.reshape(1, H), W1, b1.reshape(1, H), att0, att1)


def _run_proj_b(feat2, W2, b2, att1):
    return pl.pallas_call(
        _proj_b_body,
        grid=(GRID,),
        in_specs=[_ROW_D, _full((H, D)), _full((1, H)), _full((1, 2 * H))],
        out_specs=[_ROW_H, _SCORE],
        out_shape=[jax.ShapeDtypeStruct((N, H), jnp.float32),
                   jax.ShapeDtypeStruct((GRID, 1, RB), jnp.float32)],
    )(feat2, W2, b2.reshape(1, H), att1)


NCMAX = 20


def _sc_body(h_hbm, t_hbm, s_hbm, nei_hbm,
             out_hbm,
             tgt_v, src_v, idx_v, rows0_v, rows1_v, w_v, out0_v, out1_v,
             rsem0, rsem1, osem0, osem1, ssem):
    info = plsc.get_sparse_core_info()
    nw = info.num_cores * info.num_subcores
    wid = lax.axis_index("s") * info.num_cores + lax.axis_index("c")
    start = (wid * NCHUNK) // nw
    end = ((wid + 1) * NCHUNK) // nw
    nc = end - start

    pltpu.sync_copy(nei_hbm.at[pl.ds(start * GPC, NCMAX * GPC)], idx_v)
    pltpu.async_copy(t_hbm.at[pl.ds(start * C, NCMAX * C)], tgt_v, ssem)
    pltpu.async_copy(s_hbm, src_v, ssem)

    def issue_gathers(j, rows_b, sem):
        for g in range(GPC):
            pltpu.async_copy(h_hbm.at[idx_v.at[j * GPC + g]],
                             rows_b.at[pl.ds(g * IDXW, IDXW)], sem)

    def wait_gathers(j, rows_b, sem):
        for g in range(GPC):
            pltpu.make_async_copy(h_hbm.at[idx_v.at[j * GPC + g]],
                                  rows_b.at[pl.ds(g * IDXW, IDXW)],
                                  sem).wait()

    def process(j, j2, rows_b, sem, out_b, osem):
        cid = start + j
        node0 = cid * C
        tgt_chunk = tgt_v[pl.ds(j * C, C)]

        for n in range(C):
            r = j * GPC + n // (IDXW // S)
            col = (n % (IDXW // S)) * S
            iv0 = idx_v[r, pl.ds(col, 16)]
            iv1 = idx_v[r, pl.ds(col + 16, 16)]
            sv0 = plsc.load_gather(src_v, [iv0])
            sv1 = plsc.load_gather(src_v, [iv1])
            t = tgt_chunk[n]
            a0 = t + sv0
            a0 = jnp.where(a0 > 0, a0, a0 * 0.01)
            a1 = t + sv1
            a1 = jnp.where(a1 > 0, a1, a1 * 0.01)
            m = jnp.maximum(jnp.max(a0), jnp.max(a1))
            e0 = jnp.exp(a0 - m)
            e1 = jnp.exp(a1 - m)
            dv = jnp.full((16,), jnp.sum(e0) + jnp.sum(e1), jnp.float32)
            rv = 1.0 / dv
            w_v[pl.ds(n * S, 16)] = e0 * rv
            w_v[pl.ds(n * S + 16, 16)] = e1 * rv

        @pl.when(j2 > 0)
        def _():
            pltpu.make_async_copy(
                out_b, out_hbm.at[pl.ds((cid - 2) * C, C)], osem).wait()

        wait_gathers(j, rows_b, sem)

        def abody(n, _):
            base = n * S
            wv0 = w_v[pl.ds(base, 16)]
            wv1 = w_v[pl.ds(base + 16, 16)]
            accs = [jnp.zeros((16,), jnp.float32) for _ in range(H // 16)]
            for s in range(S):
                w = wv0[s] if s < 16 else wv1[s - 16]
                for k in range(H // 16):
                    accs[k] = accs[k] + w * rows_b[base + s, pl.ds(k * 16, 16)]
            for k in range(H // 16):
                out_b[n, pl.ds(k * 16, 16)] = accs[k]
            return 0

        lax.fori_loop(0, C, abody, 0)
        pltpu.async_copy(out_b, out_hbm.at[pl.ds(node0, C)], osem)

        @pl.when(j + 2 < nc)
        def _():
            issue_gathers(j + 2, rows_b, sem)

    issue_gathers(0, rows0_v, rsem0)
    issue_gathers(1, rows1_v, rsem1)
    pltpu.make_async_copy(
        t_hbm.at[pl.ds(start * C, NCMAX * C)], tgt_v, ssem).wait()
    pltpu.make_async_copy(s_hbm, src_v, ssem).wait()

    def pair_body(j2, _):
        process(2 * j2, j2, rows0_v, rsem0, out0_v, osem0)

        @pl.when(2 * j2 + 1 < nc)
        def _():
            process(2 * j2 + 1, j2, rows1_v, rsem1, out1_v, osem1)

        return 0

    lax.fori_loop(0, (nc + 1) // 2, pair_body, 0)

    je = ((nc - 1) // 2) * 2
    jo = ((nc - 2) // 2) * 2 + 1
    pltpu.make_async_copy(
        out0_v, out_hbm.at[pl.ds((start + je) * C, C)], osem0).wait()
    pltpu.make_async_copy(
        out1_v, out_hbm.at[pl.ds((start + jo) * C, C)], osem1).wait()


def _run_sc(h, t, s, neif):
    mesh = plsc.VectorSubcoreMesh(core_axis_name="c", subcore_axis_name="s")
    fn = pl.kernel(
        _sc_body,
        mesh=mesh,
        compiler_params=pltpu.CompilerParams(needs_layout_passes=False,
                                             use_tc_tiling_on_sc=False),
        out_type=jax.ShapeDtypeStruct((N, WOUT), jnp.float32),
        scratch_types=[
            pltpu.VMEM((NCMAX * C,), jnp.float32),
            pltpu.VMEM((N,), jnp.float32),
            pltpu.VMEM((NCMAX * GPC, IDXW), jnp.int32),
            pltpu.VMEM((BROWS, H), jnp.float32),
            pltpu.VMEM((BROWS, H), jnp.float32),
            pltpu.VMEM((BROWS,), jnp.float32),
            pltpu.VMEM((C, WOUT), jnp.float32),
            pltpu.VMEM((C, WOUT), jnp.float32),
            pltpu.SemaphoreType.DMA,
            pltpu.SemaphoreType.DMA,
            pltpu.SemaphoreType.DMA,
            pltpu.SemaphoreType.DMA,
            pltpu.SemaphoreType.DMA,
        ],
    )
    return fn(h, t, s, neif)


def _fuse_s_body(vs_r, fw_r, fb_r, es_o, ssum_o, sums):
    i = pl.program_id(0)
    es = _elu(vs_r[:, :H])
    es_o[...] = es
    part = jnp.sum(
        jnp.tanh(lax.dot_general(es, fw_r[...], (((1,), (1,)), ((), ())),
                                 preferred_element_type=jnp.float32)
                 + fb_r[...]), axis=0, keepdims=True)
    prev = jnp.where(i == 0, jnp.zeros_like(part), sums[...])
    sums[...] = prev + part

    @pl.when(i == GRID - 1)
    def _():
        ssum_o[...] = sums[...]


def _run_fuse_s(vs, fuse_W, fuse_b):
    roww = pl.BlockSpec((RB, WOUT), lambda i: (i, 0))
    return pl.pallas_call(
        _fuse_s_body,
        grid=(GRID,),
        in_specs=[roww, _full((H, H)), _full((1, H))],
        out_specs=[_ROW_H, pl.BlockSpec((1, H), lambda i: (0, 0))],
        out_shape=[jax.ShapeDtypeStruct((N, H), jnp.float32),
                   jax.ShapeDtypeStruct((1, H), jnp.float32)],
        scratch_shapes=[pltpu.VMEM((1, H), jnp.float32)],
    )(vs, fuse_W, fuse_b.reshape(1, H))


def _fuse_t_body(vt_r, es_r, fw_r, fb_r, fa_r, ssum_r, z_o,
                 sums, beta_s, et_s):
    i = pl.program_id(0)

    @pl.when(i < GRID)
    def _():
        et = _elu(vt_r[:, :H])
        et_s[pl.ds(i * RB, RB)] = et
        z_o[...] = et
        part = jnp.sum(
            jnp.tanh(lax.dot_general(et, fw_r[...], (((1,), (1,)), ((), ())),
                                     preferred_element_type=jnp.float32)
                     + fb_r[...]), axis=0, keepdims=True)
        prev = jnp.where(i == 0, jnp.zeros_like(part), sums[...])
        sums[...] = prev + part

        @pl.when(i == GRID - 1)
        def _():
            sp = jnp.concatenate([sums[...], ssum_r[...]], axis=0) * (1.0 / N)
            b2 = jnp.sum(sp * fa_r[...], axis=1, keepdims=True)
            e = jnp.exp(b2 - jnp.max(b2))
            beta_s[...] = e / jnp.sum(e)

    @pl.when(i >= GRID)
    def _():
        j = i - GRID
        et = et_s[pl.ds(j * RB, RB)]
        z_o[...] = beta_s[0, 0] * et + beta_s[1, 0] * es_r[...]


def _run_fuse_t(vt, es, fuse_W, fuse_b, fuse_att, ssum):
    roww = pl.BlockSpec((RB, WOUT),
                        lambda i: (jnp.minimum(i, GRID - 1), 0))
    esrow = pl.BlockSpec((RB, H),
                         lambda i: (jnp.maximum(i - GRID, 0), 0))
    zrow = pl.BlockSpec((RB, H), lambda i: (i % GRID, 0))
    return pl.pallas_call(
        _fuse_t_body,
        grid=(2 * GRID,),
        in_specs=[roww, esrow, _full((H, H)), _full((1, H)), _full((1, H)),
                  _full((1, H))],
        out_specs=zrow,
        out_shape=jax.ShapeDtypeStruct((N, H), jnp.float32),
        scratch_shapes=[pltpu.VMEM((1, H), jnp.float32),
                        pltpu.VMEM((2, 1), jnp.float32),
                        pltpu.VMEM((N, H), jnp.float32)],
    )(vt, es, fuse_W, fuse_b.reshape(1, H), fuse_att, ssum)


@jax.jit
def kernel(feat0, feat1, feat2, nei0, nei1, W0, b0, W1, b1, W2, b2,
           att0, att1, fuse_W, fuse_b, fuse_att):
    nei0f = jnp.reshape(nei0.astype(jnp.int32), (NEIROWS, IDXW))
    nei1f = jnp.reshape(nei1.astype(jnp.int32), (NEIROWS, IDXW))

    h1, t0, s1, t1 = _run_proj_a(feat0, feat1, W0, b0, W1, b1, att0, att1)
    flat = lambda x: x.reshape(N)

    vs = _run_sc(h1, flat(t0), flat(s1), nei0f)

    h2, s2 = _run_proj_b(feat2, W2, b2, att1)
    vt = _run_sc(h2, flat(t1), flat(s2), nei1f)

    es, ssum = _run_fuse_s(vs, fuse_W, fuse_b)
    return _run_fuse_t(vt, es, fuse_W, fuse_b, fuse_att, ssum)
